# Initial kernel scaffold; baseline (speedup 1.0000x reference)
import jax
import jax.numpy as jnp
from jax.experimental import pallas as pl


def kernel(x, edge_attr, nodes_sampled, edge_index_sampled, edge_ptr, edge_src_global, edge_index, batch, W_sub1, b_sub1, W_sub2, b_sub2, W_proj, b_proj, We, W1, b1, W2, b2, eps):
    raise NotImplementedError("write your pallas kernel here")


# speedup vs baseline: 1.0109x; 1.0109x over previous
import functools

import jax
import jax.numpy as jnp
from jax.experimental import pallas as pl
from jax.experimental.pallas import tpu as pltpu

N = 10000
E = 320000
S = 40000
ES = 320000
M = 4
G = 8
D = 128
DE = 16
H = 128
L = 3

_POOL_BLK = 1024


def _pool_body(h_ref, batch_ref, out_ref, acc_ref, cnt_ref):
    i = pl.program_id(0)

    @pl.when(i == 0)
    def _init():
        acc_ref[...] = jnp.zeros_like(acc_ref)
        cnt_ref[...] = jnp.zeros_like(cnt_ref)

    rows = h_ref[...]
    b = batch_ref[...]
    gids = jax.lax.broadcasted_iota(jnp.int32, (G, _POOL_BLK), 0)
    oh = jnp.where(gids == b, 1.0, 0.0)
    acc_ref[...] += jnp.dot(oh, rows, preferred_element_type=jnp.float32)
    cnt_ref[...] += jnp.sum(oh, axis=1, keepdims=True)

    @pl.when(i == pl.num_programs(0) - 1)
    def _fin():
        out_ref[...] = acc_ref[...] / jnp.maximum(cnt_ref[...], 1.0)


def _global_pool(h, batch):
    n_pad = ((N + _POOL_BLK - 1) // _POOL_BLK) * _POOL_BLK
    h_p = jnp.pad(h, ((0, n_pad - N), (0, 0)))
    b_p = jnp.pad(batch.astype(jnp.int32), (0, n_pad - N),
                  constant_values=G).reshape(1, n_pad)
    grid = n_pad // _POOL_BLK
    return pl.pallas_call(
        _pool_body,
        grid=(grid,),
        in_specs=[
            pl.BlockSpec((_POOL_BLK, H), lambda i: (i, 0)),
            pl.BlockSpec((1, _POOL_BLK), lambda i: (0, i)),
        ],
        out_specs=pl.BlockSpec((G, H), lambda i: (0, 0)),
        out_shape=jax.ShapeDtypeStruct((G, H), jnp.float32),
        scratch_shapes=[
            pltpu.VMEM((G, H), jnp.float32),
            pltpu.VMEM((G, 1), jnp.float32),
        ],
    )(h_p, b_p)


def kernel(x, edge_attr, nodes_sampled, edge_index_sampled, edge_ptr,
           edge_src_global, edge_index, batch,
           W_sub1, b_sub1, W_sub2, b_sub2, W_proj, b_proj,
           We, W1, b1, W2, b2, eps):
    h_all = jax.nn.relu(x @ W_sub1 + b_sub1)
    msgs = jnp.take(h_all, edge_src_global, axis=0) \
         + jnp.take(h_all, edge_index_sampled[1], axis=0)
    seg = jnp.clip(jnp.searchsorted(edge_ptr, jnp.arange(ES), side='right') - 1, 0, S - 1)
    s_sum = jax.ops.segment_sum(msgs, seg, num_segments=S)
    cnt = jnp.clip(jax.ops.segment_sum(jnp.ones((ES,), jnp.float32), seg, num_segments=S), 1.0, None)
    s_mean = s_sum / cnt[:, None]
    tgt = jnp.take(h_all, nodes_sampled, axis=0)
    sample_emb = jax.nn.relu((s_mean + tgt) @ W_sub2 + b_sub2)
    node_emb = sample_emb.reshape(N, M, H).mean(axis=1)
    h = jax.nn.relu(jnp.concatenate([x, node_emb], axis=-1) @ W_proj + b_proj)
    src = edge_index[0]
    dst = edge_index[1]
    for l in range(L):
        e = edge_attr @ We[l]
        m = jax.nn.relu(jnp.take(h, src, axis=0) + e)
        agg = jax.ops.segment_sum(m, dst, num_segments=N)
        z = (1.0 + eps[l]) * h + agg
        h = jax.nn.relu(jax.nn.relu(z @ W1[l] + b1[l]) @ W2[l] + b2[l])
    return _global_pool(h, batch)


# speedup vs baseline: 1.1281x; 1.1159x over previous
import functools

import jax
import jax.numpy as jnp
from jax import lax
from jax.experimental import pallas as pl
from jax.experimental.pallas import tpu as pltpu
from jax.experimental.pallas import tpu_sc as plsc

N = 10000
E = 320000
S = 40000
ES = 320000
M = 4
G = 8
D = 128
DE = 16
H = 128
L = 3


_ROW_BLK = 2048


def _mm_relu_body(x_ref, w_ref, b_ref, o_ref):
    o_ref[...] = jax.nn.relu(
        jnp.dot(x_ref[...], w_ref[...], preferred_element_type=jnp.float32)
        + b_ref[...])


def _mm_relu(x, w, b):
    n = x.shape[0]
    grid = (n + _ROW_BLK - 1) // _ROW_BLK
    kin = x.shape[1]
    return pl.pallas_call(
        _mm_relu_body,
        grid=(grid,),
        in_specs=[
            pl.BlockSpec((_ROW_BLK, kin), lambda i: (i, 0)),
            pl.BlockSpec((kin, H), lambda i: (0, 0)),
            pl.BlockSpec((1, H), lambda i: (0, 0)),
        ],
        out_specs=pl.BlockSpec((_ROW_BLK, H), lambda i: (i, 0)),
        out_shape=jax.ShapeDtypeStruct((n, H), jnp.float32),
    )(x, w, b.reshape(1, H))


def _gin_mlp_body(h_ref, agg_ref, eps_ref, w1_ref, b1_ref, w2_ref, b2_ref, o_ref):
    z = (1.0 + eps_ref[0, 0]) * h_ref[...] + agg_ref[0] + agg_ref[1]
    t = jax.nn.relu(
        jnp.dot(z, w1_ref[...], preferred_element_type=jnp.float32) + b1_ref[...])
    o_ref[...] = jax.nn.relu(
        jnp.dot(t, w2_ref[...], preferred_element_type=jnp.float32) + b2_ref[...])


def _gin_mlp(h, aggp, eps_l, w1, b1, w2, b2):
    grid = (N + _ROW_BLK - 1) // _ROW_BLK
    return pl.pallas_call(
        _gin_mlp_body,
        grid=(grid,),
        in_specs=[
            pl.BlockSpec((_ROW_BLK, H), lambda i: (i, 0)),
            pl.BlockSpec((2, _ROW_BLK, H), lambda i: (0, i, 0)),
            pl.BlockSpec((1, 1), lambda i: (0, 0)),
            pl.BlockSpec((H, H), lambda i: (0, 0)),
            pl.BlockSpec((1, H), lambda i: (0, 0)),
            pl.BlockSpec((H, H), lambda i: (0, 0)),
            pl.BlockSpec((1, H), lambda i: (0, 0)),
        ],
        out_specs=pl.BlockSpec((_ROW_BLK, H), lambda i: (i, 0)),
        out_shape=jax.ShapeDtypeStruct((N, H), jnp.float32),
    )(h, aggp, eps_l.reshape(1, 1), w1, b1.reshape(1, H), w2, b2.reshape(1, H))


_EBLK = 2048


def _e_body(a_ref, w_ref, o_ref):
    o_ref[...] = jnp.dot(a_ref[...], w_ref[...],
                         preferred_element_type=jnp.float32)


def _e_matmul(edge_attr, we_l):
    grid = (E + _EBLK - 1) // _EBLK
    return pl.pallas_call(
        _e_body,
        grid=(grid,),
        in_specs=[
            pl.BlockSpec((_EBLK, DE), lambda i: (i, 0)),
            pl.BlockSpec((DE, H), lambda i: (0, 0)),
        ],
        out_specs=pl.BlockSpec((_EBLK, H), lambda i: (i, 0)),
        out_shape=jax.ShapeDtypeStruct((E, H), jnp.float32),
    )(edge_attr, we_l)


_NBLK = 512


def _stage1_mlp_body(pre_ref, x_ref, ws2_ref, bs2_ref, wpt_ref, wpb_ref,
                     bp_ref, o_ref):
    se = jax.nn.relu(
        jnp.dot(pre_ref[...], ws2_ref[...], preferred_element_type=jnp.float32)
        + bs2_ref[...])
    ne = se.reshape(_NBLK, M, H).mean(axis=1)
    o_ref[...] = jax.nn.relu(
        jnp.dot(x_ref[...], wpt_ref[...], preferred_element_type=jnp.float32)
        + jnp.dot(ne, wpb_ref[...], preferred_element_type=jnp.float32)
        + bp_ref[...])


def _stage1_mlp(pre, x, ws2, bs2, wp, bp):
    grid = (N + _NBLK - 1) // _NBLK
    wpt = wp[:D]
    wpb = wp[D:]
    return pl.pallas_call(
        _stage1_mlp_body,
        grid=(grid,),
        in_specs=[
            pl.BlockSpec((M * _NBLK, H), lambda i: (i, 0)),
            pl.BlockSpec((_NBLK, D), lambda i: (i, 0)),
            pl.BlockSpec((H, H), lambda i: (0, 0)),
            pl.BlockSpec((1, H), lambda i: (0, 0)),
            pl.BlockSpec((D, H), lambda i: (0, 0)),
            pl.BlockSpec((H, H), lambda i: (0, 0)),
            pl.BlockSpec((1, H), lambda i: (0, 0)),
        ],
        out_specs=pl.BlockSpec((_NBLK, H), lambda i: (i, 0)),
        out_shape=jax.ShapeDtypeStruct((N, H), jnp.float32),
    )(pre, x, ws2, bs2.reshape(1, H), wpt, wpb, bp.reshape(1, H))


_POOL_BLK = 1024


def _pool_body(h_ref, batch_ref, out_ref, acc_ref, cnt_ref):
    i = pl.program_id(0)

    @pl.when(i == 0)
    def _init():
        acc_ref[...] = jnp.zeros_like(acc_ref)
        cnt_ref[...] = jnp.zeros_like(cnt_ref)

    rows = h_ref[...]
    b = batch_ref[...]
    gids = lax.broadcasted_iota(jnp.int32, (G, _POOL_BLK), 0)
    oh = jnp.where(gids == b, 1.0, 0.0)
    acc_ref[...] += jnp.dot(oh, rows, preferred_element_type=jnp.float32)
    cnt_ref[...] += jnp.sum(oh, axis=1, keepdims=True)

    @pl.when(i == pl.num_programs(0) - 1)
    def _fin():
        out_ref[...] = acc_ref[...] / jnp.maximum(cnt_ref[...], 1.0)


def _global_pool(h, batch):
    n_pad = ((N + _POOL_BLK - 1) // _POOL_BLK) * _POOL_BLK
    h_p = jnp.pad(h, ((0, n_pad - N), (0, 0)))
    b_p = jnp.pad(batch.astype(jnp.int32), (0, n_pad - N),
                  constant_values=G).reshape(1, n_pad)
    grid = n_pad // _POOL_BLK
    return pl.pallas_call(
        _pool_body,
        grid=(grid,),
        in_specs=[
            pl.BlockSpec((_POOL_BLK, H), lambda i: (i, 0)),
            pl.BlockSpec((1, _POOL_BLK), lambda i: (0, i)),
        ],
        out_specs=pl.BlockSpec((G, H), lambda i: (0, 0)),
        out_shape=jax.ShapeDtypeStruct((G, H), jnp.float32),
        scratch_shapes=[
            pltpu.VMEM((G, H), jnp.float32),
            pltpu.VMEM((G, 1), jnp.float32),
        ],
    )(h_p, b_p)


_NC, _NS = 2, 16
_CHUNK = 128
_NCHUNK = E // _CHUNK
_CPT = (_NCHUNK + _NC * _NS - 1) // (_NC * _NS)
_NRB = (N + _CHUNK - 1) // _CHUNK
_RBPT = (_NRB + _NS - 1) // _NS
_NRB_FULL = N // _CHUNK


def _sc_gin_body(h_hbm, e_hbm, src_hbm, dst_hbm, out_hbm,
                 src_v, dst_v, hbuf, ebuf, agg):
    c = lax.axis_index("c")
    s = lax.axis_index("s")
    w = c * _NS + s

    @pl.loop(0, _CHUNK)
    def _z(r):
        for u in range(8):
            hbuf[r, pl.ds(u * 16, 16)] = jnp.zeros((16,), jnp.float32)

    tail = N - _NRB_FULL * _CHUNK

    @pl.loop(0, _RBPT)
    def _zb(k):
        z = s * _RBPT + k

        @pl.when(z < _NRB_FULL)
        def _full():
            pltpu.sync_copy(hbuf, agg.at[pl.ds(z * _CHUNK, _CHUNK)])

        @pl.when(z == _NRB_FULL)
        def _tail():
            pltpu.sync_copy(hbuf.at[pl.ds(0, tail)],
                            agg.at[pl.ds(_NRB_FULL * _CHUNK, tail)])

    plsc.subcore_barrier()

    @pl.loop(0, _CPT)
    def _chunk(ci):
        g = w * _CPT + ci

        @pl.when(g < _NCHUNK)
        def _():
            base = g * _CHUNK
            pltpu.sync_copy(src_hbm.at[pl.ds(base, _CHUNK)], src_v)
            pltpu.sync_copy(dst_hbm.at[g], dst_v.at[0])
            pltpu.sync_copy(h_hbm.at[src_v], hbuf)
            pltpu.sync_copy(e_hbm.at[pl.ds(base, _CHUNK)], ebuf)

            @pl.loop(0, _CHUNK)
            def _row(r):
                for u in range(8):
                    sl = pl.ds(u * 16, 16)
                    ebuf[r, sl] = jnp.maximum(ebuf[r, sl] + hbuf[r, sl], 0.0)

            pltpu.sync_copy(ebuf, agg.at[dst_v.at[0]], add=True)

    plsc.subcore_barrier()

    @pl.loop(0, _RBPT)
    def _wb(k):
        z = s * _RBPT + k

        @pl.when(z < _NRB_FULL)
        def _full():
            pltpu.sync_copy(agg.at[pl.ds(z * _CHUNK, _CHUNK)],
                            out_hbm.at[c].at[pl.ds(z * _CHUNK, _CHUNK)])

        @pl.when(z == _NRB_FULL)
        def _tail():
            pltpu.sync_copy(agg.at[pl.ds(_NRB_FULL * _CHUNK, tail)],
                            out_hbm.at[c].at[pl.ds(_NRB_FULL * _CHUNK, tail)])


@functools.partial(
    pl.kernel,
    out_type=jax.ShapeDtypeStruct((_NC, N, H), jnp.float32),
    mesh=plsc.VectorSubcoreMesh(core_axis_name="c", subcore_axis_name="s"),
    scratch_types=[
        pltpu.VMEM((_CHUNK,), jnp.int32),
        pltpu.VMEM((1, _CHUNK), jnp.int32),
        pltpu.VMEM((_CHUNK, H), jnp.float32),
        pltpu.VMEM((_CHUNK, H), jnp.float32),
        pltpu.VMEM_SHARED((N, H), jnp.float32),
    ],
)
def _sc_gin(h_hbm, e_hbm, src_hbm, dst_hbm, out_hbm,
            src_v, dst_v, hbuf, ebuf, agg):
    _sc_gin_body(h_hbm, e_hbm, src_hbm, dst_hbm, out_hbm,
                 src_v, dst_v, hbuf, ebuf, agg)


def kernel(x, edge_attr, nodes_sampled, edge_index_sampled, edge_ptr,
           edge_src_global, edge_index, batch,
           W_sub1, b_sub1, W_sub2, b_sub2, W_proj, b_proj,
           We, W1, b1, W2, b2, eps):
    h_all = _mm_relu(x, W_sub1, b_sub1)
    msgs = jnp.take(h_all, edge_src_global, axis=0) \
         + jnp.take(h_all, edge_index_sampled[1], axis=0)
    seg = jnp.clip(jnp.searchsorted(edge_ptr, jnp.arange(ES), side='right') - 1,
                   0, S - 1)
    s_sum = jax.ops.segment_sum(msgs, seg, num_segments=S)
    cnt = jnp.clip(jax.ops.segment_sum(jnp.ones((ES,), jnp.float32), seg,
                                       num_segments=S), 1.0, None)
    tgt = jnp.take(h_all, nodes_sampled, axis=0)
    pre = s_sum / cnt[:, None] + tgt
    h = _stage1_mlp(pre, x, W_sub2, b_sub2, W_proj, b_proj)

    src = edge_index[0].astype(jnp.int32)
    dst2d = edge_index[1].astype(jnp.int32).reshape(_NCHUNK, _CHUNK)
    for l in range(L):
        e_l = _e_matmul(edge_attr, We[l])
        aggp = _sc_gin(h, e_l, src, dst2d)
        h = _gin_mlp(h, aggp, eps[l], W1[l], b1[l], W2[l], b2[l])

    return _global_pool(h, batch)


# speedup vs baseline: 12.4122x; 11.0029x over previous
import dataclasses
import functools

import jax
import jax.numpy as jnp
from jax import lax
from jax.experimental import pallas as pl
from jax.experimental.pallas import tpu as pltpu
from jax.experimental.pallas import tpu_sc as plsc

N = 10000
E = 320000
S = 40000
ES = 320000
M = 4
G = 8
D = 128
DE = 16
H = 128
L = 3


_ROW_BLK = 2048


def _mm_relu_body(x_ref, w_ref, b_ref, o_ref):
    o_ref[...] = jax.nn.relu(
        jnp.dot(x_ref[...], w_ref[...], preferred_element_type=jnp.float32)
        + b_ref[...])


def _mm_relu(x, w, b):
    n = x.shape[0]
    grid = (n + _ROW_BLK - 1) // _ROW_BLK
    kin = x.shape[1]
    return pl.pallas_call(
        _mm_relu_body,
        grid=(grid,),
        in_specs=[
            pl.BlockSpec((_ROW_BLK, kin), lambda i: (i, 0)),
            pl.BlockSpec((kin, H), lambda i: (0, 0)),
            pl.BlockSpec((1, H), lambda i: (0, 0)),
        ],
        out_specs=pl.BlockSpec((_ROW_BLK, H), lambda i: (i, 0)),
        out_shape=jax.ShapeDtypeStruct((n, H), jnp.float32),
    )(x, w, b.reshape(1, H))


def _gin_mlp_body(h_ref, agg_ref, eps_ref, w1_ref, b1_ref, w2_ref, b2_ref, o_ref):
    z = (1.0 + eps_ref[0, 0]) * h_ref[...] + agg_ref[0] + agg_ref[1]
    t = jax.nn.relu(
        jnp.dot(z, w1_ref[...], preferred_element_type=jnp.float32) + b1_ref[...])
    o_ref[...] = jax.nn.relu(
        jnp.dot(t, w2_ref[...], preferred_element_type=jnp.float32) + b2_ref[...])


def _gin_mlp(h, aggp, eps_l, w1, b1, w2, b2):
    grid = (N + _ROW_BLK - 1) // _ROW_BLK
    return pl.pallas_call(
        _gin_mlp_body,
        grid=(grid,),
        in_specs=[
            pl.BlockSpec((_ROW_BLK, H), lambda i: (i, 0)),
            pl.BlockSpec((2, _ROW_BLK, H), lambda i: (0, i, 0)),
            pl.BlockSpec((1, 1), lambda i: (0, 0)),
            pl.BlockSpec((H, H), lambda i: (0, 0)),
            pl.BlockSpec((1, H), lambda i: (0, 0)),
            pl.BlockSpec((H, H), lambda i: (0, 0)),
            pl.BlockSpec((1, H), lambda i: (0, 0)),
        ],
        out_specs=pl.BlockSpec((_ROW_BLK, H), lambda i: (i, 0)),
        out_shape=jax.ShapeDtypeStruct((N, H), jnp.float32),
    )(h, aggp, eps_l.reshape(1, 1), w1, b1.reshape(1, H), w2, b2.reshape(1, H))


_EBLK = 2048


def _e_body(a_ref, w_ref, o_ref):
    o_ref[...] = jnp.dot(a_ref[...], w_ref[...],
                         preferred_element_type=jnp.float32)


def _e_matmul(edge_attr, we_l):
    grid = (E + _EBLK - 1) // _EBLK
    return pl.pallas_call(
        _e_body,
        grid=(grid,),
        in_specs=[
            pl.BlockSpec((_EBLK, DE), lambda i: (i, 0)),
            pl.BlockSpec((DE, H), lambda i: (0, 0)),
        ],
        out_specs=pl.BlockSpec((_EBLK, H), lambda i: (i, 0)),
        out_shape=jax.ShapeDtypeStruct((E, H), jnp.float32),
    )(edge_attr, we_l)


_NBLK = 512


def _stage1_mlp_body(ssum_ref, tg_ref, st_ref, en_ref, x_ref,
                     ws2_ref, bs2_ref, wpt_ref, wpb_ref, bp_ref, o_ref):
    psum = ssum_ref[0] + ssum_ref[1]
    cnt = jnp.maximum((en_ref[...] - st_ref[...]).astype(jnp.float32), 1.0)
    pre = psum / cnt + tg_ref[...]
    se = jax.nn.relu(
        jnp.dot(pre, ws2_ref[...], preferred_element_type=jnp.float32)
        + bs2_ref[...])
    ne = se.reshape(_NBLK, M, H).mean(axis=1)
    o_ref[...] = jax.nn.relu(
        jnp.dot(x_ref[...], wpt_ref[...], preferred_element_type=jnp.float32)
        + jnp.dot(ne, wpb_ref[...], preferred_element_type=jnp.float32)
        + bp_ref[...])


def _stage1_mlp(ssum, tg, starts, ends, x, ws2, bs2, wp, bp):
    grid = (N + _NBLK - 1) // _NBLK
    wpt = wp[:D]
    wpb = wp[D:]
    sb = M * _NBLK
    return pl.pallas_call(
        _stage1_mlp_body,
        grid=(grid,),
        in_specs=[
            pl.BlockSpec((2, sb, H), lambda i: (0, i, 0)),
            pl.BlockSpec((sb, H), lambda i: (i, 0)),
            pl.BlockSpec((sb, 1), lambda i: (i, 0)),
            pl.BlockSpec((sb, 1), lambda i: (i, 0)),
            pl.BlockSpec((_NBLK, D), lambda i: (i, 0)),
            pl.BlockSpec((H, H), lambda i: (0, 0)),
            pl.BlockSpec((1, H), lambda i: (0, 0)),
            pl.BlockSpec((D, H), lambda i: (0, 0)),
            pl.BlockSpec((H, H), lambda i: (0, 0)),
            pl.BlockSpec((1, H), lambda i: (0, 0)),
        ],
        out_specs=pl.BlockSpec((_NBLK, H), lambda i: (i, 0)),
        out_shape=jax.ShapeDtypeStruct((N, H), jnp.float32),
    )(ssum, tg, starts, ends, x, ws2, bs2.reshape(1, H), wpt, wpb,
      bp.reshape(1, H))


_POOL_BLK = 1024


def _pool_body(h_ref, batch_ref, out_ref, acc_ref, cnt_ref):
    i = pl.program_id(0)

    @pl.when(i == 0)
    def _init():
        acc_ref[...] = jnp.zeros_like(acc_ref)
        cnt_ref[...] = jnp.zeros_like(cnt_ref)

    rows = h_ref[...]
    b = batch_ref[...]
    gids = lax.broadcasted_iota(jnp.int32, (G, _POOL_BLK), 0)
    oh = jnp.where(gids == b, 1.0, 0.0)
    acc_ref[...] += jnp.dot(oh, rows, preferred_element_type=jnp.float32)
    cnt_ref[...] += jnp.sum(oh, axis=1, keepdims=True)

    @pl.when(i == pl.num_programs(0) - 1)
    def _fin():
        out_ref[...] = acc_ref[...] / jnp.maximum(cnt_ref[...], 1.0)


def _global_pool(h, batch):
    n_pad = ((N + _POOL_BLK - 1) // _POOL_BLK) * _POOL_BLK
    h_p = jnp.pad(h, ((0, n_pad - N), (0, 0)))
    b_p = jnp.pad(batch.astype(jnp.int32), (0, n_pad - N),
                  constant_values=G).reshape(1, n_pad)
    grid = n_pad // _POOL_BLK
    return pl.pallas_call(
        _pool_body,
        grid=(grid,),
        in_specs=[
            pl.BlockSpec((_POOL_BLK, H), lambda i: (i, 0)),
            pl.BlockSpec((1, _POOL_BLK), lambda i: (0, i)),
        ],
        out_specs=pl.BlockSpec((G, H), lambda i: (0, 0)),
        out_shape=jax.ShapeDtypeStruct((G, H), jnp.float32),
        scratch_shapes=[
            pltpu.VMEM((G, H), jnp.float32),
            pltpu.VMEM((G, 1), jnp.float32),
        ],
    )(h_p, b_p)


_NC, _NS = 2, 16
_CHUNK = 128
_NCHUNK = E // _CHUNK
_CPT = (_NCHUNK + _NC * _NS - 1) // (_NC * _NS)
_NRB = (N + _CHUNK - 1) // _CHUNK
_RBPT = (_NRB + _NS - 1) // _NS
_NRB_FULL = N // _CHUNK


def _sc_gin_body(h_hbm, e_hbm, src_hbm, dst_hbm, out_hbm,
                 src_v, dst_v, hbuf, ebuf, agg):
    c = lax.axis_index("c")
    s = lax.axis_index("s")
    w = c * _NS + s

    @pl.loop(0, _CHUNK)
    def _z(r):
        for u in range(8):
            hbuf[r, pl.ds(u * 16, 16)] = jnp.zeros((16,), jnp.float32)

    tail = N - _NRB_FULL * _CHUNK

    @pl.loop(0, _RBPT)
    def _zb(k):
        z = s * _RBPT + k

        @pl.when(z < _NRB_FULL)
        def _full():
            pltpu.sync_copy(hbuf, agg.at[pl.ds(z * _CHUNK, _CHUNK)])

        @pl.when(z == _NRB_FULL)
        def _tail():
            pltpu.sync_copy(hbuf.at[pl.ds(0, tail)],
                            agg.at[pl.ds(_NRB_FULL * _CHUNK, tail)])

    plsc.subcore_barrier()

    @pl.loop(0, _CPT)
    def _chunk(ci):
        g = w * _CPT + ci

        @pl.when(g < _NCHUNK)
        def _():
            base = g * _CHUNK
            pltpu.sync_copy(src_hbm.at[pl.ds(base, _CHUNK)], src_v)
            pltpu.sync_copy(dst_hbm.at[g], dst_v.at[0])
            pltpu.sync_copy(h_hbm.at[src_v], hbuf)
            pltpu.sync_copy(e_hbm.at[pl.ds(base, _CHUNK)], ebuf)

            @pl.loop(0, _CHUNK)
            def _row(r):
                for u in range(8):
                    sl = pl.ds(u * 16, 16)
                    ebuf[r, sl] = jnp.maximum(ebuf[r, sl] + hbuf[r, sl], 0.0)

            pltpu.sync_copy(ebuf, agg.at[dst_v.at[0]], add=True)

    plsc.subcore_barrier()

    @pl.loop(0, _RBPT)
    def _wb(k):
        z = s * _RBPT + k

        @pl.when(z < _NRB_FULL)
        def _full():
            pltpu.sync_copy(agg.at[pl.ds(z * _CHUNK, _CHUNK)],
                            out_hbm.at[c].at[pl.ds(z * _CHUNK, _CHUNK)])

        @pl.when(z == _NRB_FULL)
        def _tail():
            pltpu.sync_copy(agg.at[pl.ds(_NRB_FULL * _CHUNK, tail)],
                            out_hbm.at[c].at[pl.ds(_NRB_FULL * _CHUNK, tail)])


@functools.partial(
    pl.kernel,
    out_type=jax.ShapeDtypeStruct((_NC, N, H), jnp.float32),
    mesh=plsc.VectorSubcoreMesh(core_axis_name="c", subcore_axis_name="s"),
    scratch_types=[
        pltpu.VMEM((_CHUNK,), jnp.int32),
        pltpu.VMEM((1, _CHUNK), jnp.int32),
        pltpu.VMEM((_CHUNK, H), jnp.float32),
        pltpu.VMEM((_CHUNK, H), jnp.float32),
        pltpu.VMEM_SHARED((N, H), jnp.float32),
    ],
)
def _sc_gin(h_hbm, e_hbm, src_hbm, dst_hbm, out_hbm,
            src_v, dst_v, hbuf, ebuf, agg):
    _sc_gin_body(h_hbm, e_hbm, src_hbm, dst_hbm, out_hbm,
                 src_v, dst_v, hbuf, ebuf, agg)


_SC_CP = pltpu.CompilerParams()
if "needs_layout_passes" in pltpu.CompilerParams.__dataclass_fields__:
    _SC_CP = dataclasses.replace(_SC_CP, needs_layout_passes=False)

_PTR_PAD = 40016
_SBLK = 8000
_NPH = S // _SBLK
_S1_NFULL = _SBLK // _CHUNK
_S1_TAIL = _SBLK - _S1_NFULL * _CHUNK
_S1_NRB = _S1_NFULL + 1
_S1_RBPT = (_S1_NRB + _NS - 1) // _NS
_ECPT = _CPT
_TGC = (S + 64) // _CHUNK
_TGPT = (_TGC + _NC * _NS - 1) // (_NC * _NS)
_SEGLEN = _ECPT * _CHUNK


def _count_lt(ptrv, x):
    iota = lax.iota(jnp.int32, 16)
    base = jnp.int32(0)
    for stride in (2501, 157, 10, 1):
        idx = jnp.minimum(base + (iota + 1) * stride - 1, _PTR_PAD - 1)
        vals = plsc.load_gather(ptrv, [idx])
        cnt = jnp.sum((vals < x).astype(jnp.int32))
        base = base + cnt * stride
    return base


def _sc_stage1_body(h_hbm, esg_hbm, eis_hbm, ptr_hbm, ns_hbm,
                    ssum_hbm, tg_hbm,
                    ptrv, seg1d, adj, iv, abuf, agg):
    c = lax.axis_index("c")
    s = lax.axis_index("s")
    w = c * _NS + s
    iota = lax.iota(jnp.int32, 16)

    pltpu.sync_copy(ptr_hbm, ptrv)

    @pl.loop(0, _TGPT)
    def _tg(k):
        g = w * _TGPT + k

        @pl.when(g < _TGC)
        def _():
            pltpu.sync_copy(ns_hbm.at[g], iv)
            pltpu.sync_copy(h_hbm.at[iv], abuf)
            pltpu.sync_copy(abuf, tg_hbm.at[pl.ds(g * _CHUNK, _CHUNK)])

    b0 = w * _SEGLEN
    b1 = jnp.minimum(b0 + _SEGLEN, ES)
    sa = _count_lt(ptrv, b0)
    sb = _count_lt(ptrv, b1)
    seed = jnp.clip(sa - 1, 0, S - 1)

    @pl.loop(0, _ECPT)
    def _zs(ci):
        for u in range(8):
            seg1d[pl.ds(ci * _CHUNK + u * 16, 16)] = jnp.zeros((16,), jnp.int32)

    ngrp = (sb - sa + 15) // 16

    def _scatter_starts(gi, _):
        t16 = sa + gi * 16 + iota
        tc = jnp.minimum(t16, _PTR_PAD - 1)
        vals = plsc.load_gather(ptrv, [tc])
        nxt = plsc.load_gather(ptrv, [jnp.minimum(tc + 1, _PTR_PAD - 1)])
        pos = vals - b0
        keep = ((t16 < sb) & (vals != nxt)
                & (pos >= 0) & (pos < _SEGLEN))
        plsc.store_scatter(seg1d, [jnp.where(keep, pos, 0)],
                           jnp.minimum(t16, S - 1), mask=keep)
        return 0

    lax.fori_loop(0, ngrp, _scatter_starts, 0)

    def _sweep(gi, carry):
        v = seg1d[pl.ds(gi * 16, 16)]
        v = jnp.maximum(plsc.cummax(v), carry)
        seg1d[pl.ds(gi * 16, 16)] = v
        return jnp.full((16,), jnp.max(v), jnp.int32)

    lax.fori_loop(0, _SEGLEN // 16, _sweep,
                  jnp.full((16,), seed, jnp.int32))

    @pl.loop(0, _CHUNK)
    def _zb0(r):
        for u in range(8):
            abuf.at[r][pl.ds(u * 16, 16)] = jnp.zeros((16,), jnp.float32)

    tail = _S1_TAIL
    nfull = _S1_NFULL

    for ph in range(_NPH):
        blo = ph * _SBLK

        @pl.loop(0, _S1_RBPT)
        def _zb(k):
            z = s * _S1_RBPT + k

            @pl.when(z < nfull)
            def _full():
                pltpu.sync_copy(abuf, agg.at[pl.ds(z * _CHUNK, _CHUNK)])

            @pl.when(z == nfull)
            def _tl():
                pltpu.sync_copy(abuf.at[pl.ds(0, tail)],
                                agg.at[pl.ds(nfull * _CHUNK, tail)])

        plsc.subcore_barrier()

        @pl.loop(0, _ECPT)
        def _chunk(ci):
            g = w * _ECPT + ci

            @pl.when(g < _NCHUNK)
            def _():
                lo16 = seg1d[pl.ds(ci * _CHUNK, 16)]
                hi16 = seg1d[pl.ds(ci * _CHUNK + _CHUNK - 16, 16)]
                cmin = jnp.min(lo16)
                cmax = jnp.max(hi16)

                @pl.when((cmax >= blo) & (cmin < blo + _SBLK))
                def _active():
                    adj_row = adj.at[0]
                    for u in range(8):
                        v = seg1d[pl.ds(ci * _CHUNK + u * 16, 16)] - blo
                        inb = (v >= 0) & (v < _SBLK)
                        adj_row[pl.ds(u * 16, 16)] = jnp.where(inb, v, _SBLK)
                    base = g * _CHUNK
                    pltpu.sync_copy(esg_hbm.at[pl.ds(base, _CHUNK)], iv)
                    pltpu.sync_copy(h_hbm.at[iv], abuf)
                    pltpu.sync_copy(abuf, agg.at[adj.at[0]], add=True)
                    pltpu.sync_copy(eis_hbm.at[pl.ds(base, _CHUNK)], iv)
                    pltpu.sync_copy(h_hbm.at[iv], abuf)
                    pltpu.sync_copy(abuf, agg.at[adj.at[0]], add=True)

        plsc.subcore_barrier()

        @pl.loop(0, _S1_RBPT)
        def _wb(k):
            z = s * _S1_RBPT + k

            @pl.when(z < nfull)
            def _full():
                pltpu.sync_copy(
                    agg.at[pl.ds(z * _CHUNK, _CHUNK)],
                    ssum_hbm.at[c].at[pl.ds(blo + z * _CHUNK, _CHUNK)])

            @pl.when(z == nfull)
            def _tl():
                pltpu.sync_copy(
                    agg.at[pl.ds(nfull * _CHUNK, tail)],
                    ssum_hbm.at[c].at[pl.ds(blo + nfull * _CHUNK, tail)])

        if ph + 1 < _NPH:
            @pl.loop(0, _CHUNK)
            def _rz(r):
                for u in range(8):
                    abuf.at[r][pl.ds(u * 16, 16)] = jnp.zeros((16,), jnp.float32)


@functools.partial(
    pl.kernel,
    out_type=(jax.ShapeDtypeStruct((_NC, S, H), jnp.float32),
              jax.ShapeDtypeStruct((_TGC * _CHUNK, H), jnp.float32)),
    mesh=plsc.VectorSubcoreMesh(core_axis_name="c", subcore_axis_name="s"),
    scratch_types=[
        pltpu.VMEM((_PTR_PAD,), jnp.int32),
        pltpu.VMEM((_SEGLEN,), jnp.int32),
        pltpu.VMEM((1, _CHUNK), jnp.int32),
        pltpu.VMEM((_CHUNK,), jnp.int32),
        pltpu.VMEM((_CHUNK, H), jnp.float32),
        pltpu.VMEM_SHARED((_SBLK + 1, H), jnp.float32),
    ],
    compiler_params=_SC_CP,
)
def _sc_stage1(h_hbm, esg_hbm, eis_hbm, ptr_hbm, ns_hbm, ssum_hbm, tg_hbm,
               ptrv, seg1d, adj, iv, abuf, agg):
    _sc_stage1_body(h_hbm, esg_hbm, eis_hbm, ptr_hbm, ns_hbm,
                    ssum_hbm, tg_hbm, ptrv, seg1d, adj, iv, abuf, agg)


def kernel(x, edge_attr, nodes_sampled, edge_index_sampled, edge_ptr,
           edge_src_global, edge_index, batch,
           W_sub1, b_sub1, W_sub2, b_sub2, W_proj, b_proj,
           We, W1, b1, W2, b2, eps):
    h_all = _mm_relu(x, W_sub1, b_sub1)
    ptr32 = edge_ptr.astype(jnp.int32)
    ptr_pad = jnp.pad(ptr32, (0, _PTR_PAD - (S + 1)),
                      constant_values=jnp.int32(2**31 - 1))
    ns_pad = jnp.pad(nodes_sampled.astype(jnp.int32),
                     (0, _TGC * _CHUNK - S)).reshape(_TGC, _CHUNK)
    esg = edge_src_global.astype(jnp.int32)
    eis = edge_index_sampled[1].astype(jnp.int32)
    ssum, tg = _sc_stage1(h_all, esg, eis, ptr_pad, ns_pad)
    starts = jnp.concatenate([jnp.zeros((1,), jnp.int32),
                              ptr32[1:S]]).reshape(S, 1)
    ends = jnp.concatenate([ptr32[1:S],
                            jnp.full((1,), ES, jnp.int32)]).reshape(S, 1)
    h = _stage1_mlp(ssum, tg, starts, ends, x, W_sub2, b_sub2, W_proj, b_proj)

    src = edge_index[0].astype(jnp.int32)
    dst2d = edge_index[1].astype(jnp.int32).reshape(_NCHUNK, _CHUNK)
    for l in range(L):
        e_l = _e_matmul(edge_attr, We[l])
        aggp = _sc_gin(h, e_l, src, dst2d)
        h = _gin_mlp(h, aggp, eps[l], W1[l], b1[l], W2[l], b2[l])

    return _global_pool(h, batch)


# speedup vs baseline: 12.4190x; 1.0005x over previous
import dataclasses
import functools

import jax
import jax.numpy as jnp
from jax import lax
from jax.experimental import pallas as pl
from jax.experimental.pallas import tpu as pltpu
from jax.experimental.pallas import tpu_sc as plsc

N = 10000
E = 320000
S = 40000
ES = 320000
M = 4
G = 8
D = 128
DE = 16
H = 128
L = 3


_ROW_BLK = 2048


def _mm_relu_body(x_ref, w_ref, b_ref, o_ref):
    o_ref[...] = jax.nn.relu(
        jnp.dot(x_ref[...], w_ref[...], preferred_element_type=jnp.float32)
        + b_ref[...])


def _mm_relu(x, w, b):
    n = x.shape[0]
    grid = (n + _ROW_BLK - 1) // _ROW_BLK
    kin = x.shape[1]
    return pl.pallas_call(
        _mm_relu_body,
        grid=(grid,),
        in_specs=[
            pl.BlockSpec((_ROW_BLK, kin), lambda i: (i, 0)),
            pl.BlockSpec((kin, H), lambda i: (0, 0)),
            pl.BlockSpec((1, H), lambda i: (0, 0)),
        ],
        out_specs=pl.BlockSpec((_ROW_BLK, H), lambda i: (i, 0)),
        out_shape=jax.ShapeDtypeStruct((n, H), jnp.float32),
    )(x, w, b.reshape(1, H))


def _gin_mlp_body(h_ref, agg_ref, eps_ref, w1_ref, b1_ref, w2_ref, b2_ref, o_ref):
    z = (1.0 + eps_ref[0, 0]) * h_ref[...] + agg_ref[0] + agg_ref[1]
    t = jax.nn.relu(
        jnp.dot(z, w1_ref[...], preferred_element_type=jnp.float32) + b1_ref[...])
    o_ref[...] = jax.nn.relu(
        jnp.dot(t, w2_ref[...], preferred_element_type=jnp.float32) + b2_ref[...])


def _gin_mlp(h, aggp, eps_l, w1, b1, w2, b2):
    grid = (N + _ROW_BLK - 1) // _ROW_BLK
    return pl.pallas_call(
        _gin_mlp_body,
        grid=(grid,),
        in_specs=[
            pl.BlockSpec((_ROW_BLK, H), lambda i: (i, 0)),
            pl.BlockSpec((2, _ROW_BLK, H), lambda i: (0, i, 0)),
            pl.BlockSpec((1, 1), lambda i: (0, 0)),
            pl.BlockSpec((H, H), lambda i: (0, 0)),
            pl.BlockSpec((1, H), lambda i: (0, 0)),
            pl.BlockSpec((H, H), lambda i: (0, 0)),
            pl.BlockSpec((1, H), lambda i: (0, 0)),
        ],
        out_specs=pl.BlockSpec((_ROW_BLK, H), lambda i: (i, 0)),
        out_shape=jax.ShapeDtypeStruct((N, H), jnp.float32),
    )(h, aggp, eps_l.reshape(1, 1), w1, b1.reshape(1, H), w2, b2.reshape(1, H))


_EBLK = 2048


def _e_body(a_ref, w_ref, o_ref):
    o_ref[...] = jnp.dot(a_ref[...], w_ref[...],
                         preferred_element_type=jnp.float32)


def _e_matmul(edge_attr, we_l, n_out):
    grid = (n_out + _EBLK - 1) // _EBLK
    return pl.pallas_call(
        _e_body,
        grid=(grid,),
        in_specs=[
            pl.BlockSpec((_EBLK, DE), lambda i: (i, 0)),
            pl.BlockSpec((DE, H), lambda i: (0, 0)),
        ],
        out_specs=pl.BlockSpec((_EBLK, H), lambda i: (i, 0)),
        out_shape=jax.ShapeDtypeStruct((n_out, H), jnp.float32),
    )(edge_attr, we_l)


_NBLK = 512


def _stage1_mlp_body(ssum_ref, tg_ref, st_ref, en_ref, x_ref,
                     ws2_ref, bs2_ref, wpt_ref, wpb_ref, bp_ref, o_ref):
    psum = ssum_ref[0] + ssum_ref[1]
    cnt = jnp.maximum((en_ref[...] - st_ref[...]).astype(jnp.float32), 1.0)
    pre = psum / cnt + tg_ref[...]
    se = jax.nn.relu(
        jnp.dot(pre, ws2_ref[...], preferred_element_type=jnp.float32)
        + bs2_ref[...])
    ne = se.reshape(_NBLK, M, H).mean(axis=1)
    o_ref[...] = jax.nn.relu(
        jnp.dot(x_ref[...], wpt_ref[...], preferred_element_type=jnp.float32)
        + jnp.dot(ne, wpb_ref[...], preferred_element_type=jnp.float32)
        + bp_ref[...])


def _stage1_mlp(ssum, tg, starts, ends, x, ws2, bs2, wp, bp):
    grid = (N + _NBLK - 1) // _NBLK
    wpt = wp[:D]
    wpb = wp[D:]
    sb = M * _NBLK
    return pl.pallas_call(
        _stage1_mlp_body,
        grid=(grid,),
        in_specs=[
            pl.BlockSpec((2, sb, H), lambda i: (0, i, 0)),
            pl.BlockSpec((sb, H), lambda i: (i, 0)),
            pl.BlockSpec((sb, 1), lambda i: (i, 0)),
            pl.BlockSpec((sb, 1), lambda i: (i, 0)),
            pl.BlockSpec((_NBLK, D), lambda i: (i, 0)),
            pl.BlockSpec((H, H), lambda i: (0, 0)),
            pl.BlockSpec((1, H), lambda i: (0, 0)),
            pl.BlockSpec((D, H), lambda i: (0, 0)),
            pl.BlockSpec((H, H), lambda i: (0, 0)),
            pl.BlockSpec((1, H), lambda i: (0, 0)),
        ],
        out_specs=pl.BlockSpec((_NBLK, H), lambda i: (i, 0)),
        out_shape=jax.ShapeDtypeStruct((N, H), jnp.float32),
    )(ssum, tg, starts, ends, x, ws2, bs2.reshape(1, H), wpt, wpb,
      bp.reshape(1, H))


_POOL_BLK = 1024


def _pool_body(h_ref, batch_ref, out_ref, acc_ref, cnt_ref):
    i = pl.program_id(0)

    @pl.when(i == 0)
    def _init():
        acc_ref[...] = jnp.zeros_like(acc_ref)
        cnt_ref[...] = jnp.zeros_like(cnt_ref)

    rows = h_ref[...]
    b = batch_ref[...]
    gids = lax.broadcasted_iota(jnp.int32, (G, _POOL_BLK), 0)
    oh = jnp.where(gids == b, 1.0, 0.0)
    acc_ref[...] += jnp.dot(oh, rows, preferred_element_type=jnp.float32)
    cnt_ref[...] += jnp.sum(oh, axis=1, keepdims=True)

    @pl.when(i == pl.num_programs(0) - 1)
    def _fin():
        out_ref[...] = acc_ref[...] / jnp.maximum(cnt_ref[...], 1.0)


def _global_pool(h, batch):
    n_pad = ((N + _POOL_BLK - 1) // _POOL_BLK) * _POOL_BLK
    h_p = jnp.pad(h, ((0, n_pad - N), (0, 0)))
    b_p = jnp.pad(batch.astype(jnp.int32), (0, n_pad - N),
                  constant_values=G).reshape(1, n_pad)
    grid = n_pad // _POOL_BLK
    return pl.pallas_call(
        _pool_body,
        grid=(grid,),
        in_specs=[
            pl.BlockSpec((_POOL_BLK, H), lambda i: (i, 0)),
            pl.BlockSpec((1, _POOL_BLK), lambda i: (0, i)),
        ],
        out_specs=pl.BlockSpec((G, H), lambda i: (0, 0)),
        out_shape=jax.ShapeDtypeStruct((G, H), jnp.float32),
        scratch_shapes=[
            pltpu.VMEM((G, H), jnp.float32),
            pltpu.VMEM((G, 1), jnp.float32),
        ],
    )(h_p, b_p)


_NC, _NS = 2, 16
_CHUNK = 128
_NCHUNK = E // _CHUNK
_CPT = (_NCHUNK + _NC * _NS - 1) // (_NC * _NS)
_NRB = (N + _CHUNK - 1) // _CHUNK
_RBPT = (_NRB + _NS - 1) // _NS
_NRB_FULL = N // _CHUNK


_E_PAD = _NC * _NS * _CPT * _CHUNK
_SPT = _CPT * _CHUNK


def _sc_gin_body(h_hbm, e_hbm, src_hbm, dst_hbm, out_hbm,
                 src_v, dst_v, hbuf, ebuf, agg):
    c = lax.axis_index("c")
    s = lax.axis_index("s")
    w = c * _NS + s

    @pl.loop(0, _CHUNK)
    def _z(r):
        for u in range(8):
            hbuf[r, pl.ds(u * 16, 16)] = jnp.zeros((16,), jnp.float32)

    tail = N - _NRB_FULL * _CHUNK

    @pl.loop(0, _RBPT)
    def _zb(k):
        z = s * _RBPT + k

        @pl.when(z < _NRB_FULL)
        def _full():
            pltpu.sync_copy(hbuf, agg.at[pl.ds(z * _CHUNK, _CHUNK)])

        @pl.when(z == _NRB_FULL)
        def _tail():
            pltpu.sync_copy(hbuf.at[pl.ds(0, tail)],
                            agg.at[pl.ds(_NRB_FULL * _CHUNK, tail)])

    plsc.subcore_barrier()

    @pl.loop(0, _CPT)
    def _chunk(ci):
        g = w * _CPT + ci

        @pl.when(g < _NCHUNK)
        def _():
            base = g * _CHUNK
            pltpu.sync_copy(src_hbm.at[pl.ds(base, _CHUNK)], src_v)
            pltpu.sync_copy(dst_hbm.at[g], dst_v.at[0])
            pltpu.sync_copy(h_hbm.at[src_v], hbuf)
            pltpu.sync_copy(e_hbm.at[pl.ds(base, _CHUNK)], ebuf)

            @pl.loop(0, _CHUNK)
            def _row(r):
                for u in range(8):
                    sl = pl.ds(u * 16, 16)
                    ebuf[r, sl] = jnp.maximum(ebuf[r, sl] + hbuf[r, sl], 0.0)

            pltpu.sync_copy(ebuf, agg.at[dst_v.at[0]], add=True)

    plsc.subcore_barrier()

    @pl.loop(0, _RBPT)
    def _wb(k):
        z = s * _RBPT + k

        @pl.when(z < _NRB_FULL)
        def _full():
            pltpu.sync_copy(agg.at[pl.ds(z * _CHUNK, _CHUNK)],
                            out_hbm.at[c].at[pl.ds(z * _CHUNK, _CHUNK)])

        @pl.when(z == _NRB_FULL)
        def _tail():
            pltpu.sync_copy(agg.at[pl.ds(_NRB_FULL * _CHUNK, tail)],
                            out_hbm.at[c].at[pl.ds(_NRB_FULL * _CHUNK, tail)])


@functools.partial(
    pl.kernel,
    out_type=jax.ShapeDtypeStruct((_NC, N, H), jnp.float32),
    mesh=plsc.VectorSubcoreMesh(core_axis_name="c", subcore_axis_name="s"),
    scratch_types=[
        pltpu.VMEM((_CHUNK,), jnp.int32),
        pltpu.VMEM((1, _CHUNK), jnp.int32),
        pltpu.VMEM((_CHUNK, H), jnp.float32),
        pltpu.VMEM((_CHUNK, H), jnp.float32),
        pltpu.VMEM_SHARED((N, H), jnp.float32),
    ],
)
def _sc_gin(h_hbm, e_hbm, src_hbm, dst_hbm, out_hbm,
            src_v, dst_v, hbuf, ebuf, agg):
    _sc_gin_body(h_hbm, e_hbm, src_hbm, dst_hbm, out_hbm,
                 src_v, dst_v, hbuf, ebuf, agg)


_SC_CP = pltpu.CompilerParams()
if "needs_layout_passes" in pltpu.CompilerParams.__dataclass_fields__:
    _SC_CP = dataclasses.replace(_SC_CP, needs_layout_passes=False)

_PTR_PAD = 40016
_SBLK = 8000
_NPH = S // _SBLK
_S1_NFULL = _SBLK // _CHUNK
_S1_TAIL = _SBLK - _S1_NFULL * _CHUNK
_S1_NRB = _S1_NFULL + 1
_S1_RBPT = (_S1_NRB + _NS - 1) // _NS
_ECPT = _CPT
_TGC = (S + 64) // _CHUNK
_TGPT = (_TGC + _NC * _NS - 1) // (_NC * _NS)
_SEGLEN = _ECPT * _CHUNK


def _count_lt(ptrv, x):
    iota = lax.iota(jnp.int32, 16)
    base = jnp.int32(0)
    for stride in (2501, 157, 10, 1):
        idx = jnp.minimum(base + (iota + 1) * stride - 1, _PTR_PAD - 1)
        vals = plsc.load_gather(ptrv, [idx])
        cnt = jnp.sum((vals < x).astype(jnp.int32))
        base = base + cnt * stride
    return base


def _sc_stage1_body(h_hbm, esg_hbm, eis_hbm, ptr_hbm, ns_hbm,
                    ssum_hbm, tg_hbm,
                    ptrv, seg1d, adj, iv, abuf, agg):
    c = lax.axis_index("c")
    s = lax.axis_index("s")
    w = c * _NS + s
    iota = lax.iota(jnp.int32, 16)

    pltpu.sync_copy(ptr_hbm, ptrv)

    @pl.loop(0, _TGPT)
    def _tg(k):
        g = w * _TGPT + k

        @pl.when(g < _TGC)
        def _():
            pltpu.sync_copy(ns_hbm.at[g], iv)
            pltpu.sync_copy(h_hbm.at[iv], abuf)
            pltpu.sync_copy(abuf, tg_hbm.at[pl.ds(g * _CHUNK, _CHUNK)])

    b0 = w * _SEGLEN
    b1 = jnp.minimum(b0 + _SEGLEN, ES)
    sa = _count_lt(ptrv, b0)
    sb = _count_lt(ptrv, b1)
    seed = jnp.clip(sa - 1, 0, S - 1)

    @pl.loop(0, _ECPT)
    def _zs(ci):
        for u in range(8):
            seg1d[pl.ds(ci * _CHUNK + u * 16, 16)] = jnp.zeros((16,), jnp.int32)

    ngrp = (sb - sa + 15) // 16

    def _scatter_starts(gi, _):
        t16 = sa + gi * 16 + iota
        tc = jnp.minimum(t16, _PTR_PAD - 1)
        vals = plsc.load_gather(ptrv, [tc])
        nxt = plsc.load_gather(ptrv, [jnp.minimum(tc + 1, _PTR_PAD - 1)])
        pos = vals - b0
        keep = ((t16 < sb) & (vals != nxt)
                & (pos >= 0) & (pos < _SEGLEN))
        plsc.store_scatter(seg1d, [jnp.where(keep, pos, 0)],
                           jnp.minimum(t16, S - 1), mask=keep)
        return 0

    lax.fori_loop(0, ngrp, _scatter_starts, 0)

    def _sweep(gi, carry):
        v = seg1d[pl.ds(gi * 16, 16)]
        v = jnp.maximum(plsc.cummax(v), carry)
        seg1d[pl.ds(gi * 16, 16)] = v
        return jnp.full((16,), jnp.max(v), jnp.int32)

    lax.fori_loop(0, _SEGLEN // 16, _sweep,
                  jnp.full((16,), seed, jnp.int32))

    @pl.loop(0, _CHUNK)
    def _zb0(r):
        for u in range(8):
            abuf.at[r][pl.ds(u * 16, 16)] = jnp.zeros((16,), jnp.float32)

    tail = _S1_TAIL
    nfull = _S1_NFULL

    for ph in range(_NPH):
        blo = ph * _SBLK

        @pl.loop(0, _S1_RBPT)
        def _zb(k):
            z = s * _S1_RBPT + k

            @pl.when(z < nfull)
            def _full():
                pltpu.sync_copy(abuf, agg.at[pl.ds(z * _CHUNK, _CHUNK)])

            @pl.when(z == nfull)
            def _tl():
                pltpu.sync_copy(abuf.at[pl.ds(0, tail)],
                                agg.at[pl.ds(nfull * _CHUNK, tail)])

        plsc.subcore_barrier()

        @pl.loop(0, _ECPT)
        def _chunk(ci):
            g = w * _ECPT + ci

            @pl.when(g < _NCHUNK)
            def _():
                lo16 = seg1d[pl.ds(ci * _CHUNK, 16)]
                hi16 = seg1d[pl.ds(ci * _CHUNK + _CHUNK - 16, 16)]
                cmin = jnp.min(lo16)
                cmax = jnp.max(hi16)

                @pl.when((cmax >= blo) & (cmin < blo + _SBLK))
                def _active():
                    adj_row = adj.at[0]
                    for u in range(8):
                        v = seg1d[pl.ds(ci * _CHUNK + u * 16, 16)] - blo
                        inb = (v >= 0) & (v < _SBLK)
                        adj_row[pl.ds(u * 16, 16)] = jnp.where(inb, v, _SBLK)
                    base = g * _CHUNK
                    pltpu.sync_copy(esg_hbm.at[pl.ds(base, _CHUNK)], iv)
                    pltpu.sync_copy(h_hbm.at[iv], abuf)
                    pltpu.sync_copy(abuf, agg.at[adj.at[0]], add=True)
                    pltpu.sync_copy(eis_hbm.at[pl.ds(base, _CHUNK)], iv)
                    pltpu.sync_copy(h_hbm.at[iv], abuf)
                    pltpu.sync_copy(abuf, agg.at[adj.at[0]], add=True)

        plsc.subcore_barrier()

        @pl.loop(0, _S1_RBPT)
        def _wb(k):
            z = s * _S1_RBPT + k

            @pl.when(z < nfull)
            def _full():
                pltpu.sync_copy(
                    agg.at[pl.ds(z * _CHUNK, _CHUNK)],
                    ssum_hbm.at[c].at[pl.ds(blo + z * _CHUNK, _CHUNK)])

            @pl.when(z == nfull)
            def _tl():
                pltpu.sync_copy(
                    agg.at[pl.ds(nfull * _CHUNK, tail)],
                    ssum_hbm.at[c].at[pl.ds(blo + nfull * _CHUNK, tail)])

        if ph + 1 < _NPH:
            @pl.loop(0, _CHUNK)
            def _rz(r):
                for u in range(8):
                    abuf.at[r][pl.ds(u * 16, 16)] = jnp.zeros((16,), jnp.float32)


@functools.partial(
    pl.kernel,
    out_type=(jax.ShapeDtypeStruct((_NC, S, H), jnp.float32),
              jax.ShapeDtypeStruct((_TGC * _CHUNK, H), jnp.float32)),
    mesh=plsc.VectorSubcoreMesh(core_axis_name="c", subcore_axis_name="s"),
    scratch_types=[
        pltpu.VMEM((_PTR_PAD,), jnp.int32),
        pltpu.VMEM((_SEGLEN,), jnp.int32),
        pltpu.VMEM((1, _CHUNK), jnp.int32),
        pltpu.VMEM((_CHUNK,), jnp.int32),
        pltpu.VMEM((_CHUNK, H), jnp.float32),
        pltpu.VMEM_SHARED((_SBLK + 1, H), jnp.float32),
    ],
    compiler_params=_SC_CP,
)
def _sc_stage1(h_hbm, esg_hbm, eis_hbm, ptr_hbm, ns_hbm, ssum_hbm, tg_hbm,
               ptrv, seg1d, adj, iv, abuf, agg):
    _sc_stage1_body(h_hbm, esg_hbm, eis_hbm, ptr_hbm, ns_hbm,
                    ssum_hbm, tg_hbm, ptrv, seg1d, adj, iv, abuf, agg)


def kernel(x, edge_attr, nodes_sampled, edge_index_sampled, edge_ptr,
           edge_src_global, edge_index, batch,
           W_sub1, b_sub1, W_sub2, b_sub2, W_proj, b_proj,
           We, W1, b1, W2, b2, eps):
    h_all = _mm_relu(x, W_sub1, b_sub1)
    ptr32 = edge_ptr.astype(jnp.int32)
    ptr_pad = jnp.pad(ptr32, (0, _PTR_PAD - (S + 1)),
                      constant_values=jnp.int32(2**31 - 1))
    ns_pad = jnp.pad(nodes_sampled.astype(jnp.int32),
                     (0, _TGC * _CHUNK - S)).reshape(_TGC, _CHUNK)
    esg = edge_src_global.astype(jnp.int32)
    eis = edge_index_sampled[1].astype(jnp.int32)
    ssum, tg = _sc_stage1(h_all, esg, eis, ptr_pad, ns_pad)
    starts = jnp.concatenate([jnp.zeros((1,), jnp.int32),
                              ptr32[1:S]]).reshape(S, 1)
    ends = jnp.concatenate([ptr32[1:S],
                            jnp.full((1,), ES, jnp.int32)]).reshape(S, 1)
    h = _stage1_mlp(ssum, tg, starts, ends, x, W_sub2, b_sub2, W_proj, b_proj)

    src = edge_index[0].astype(jnp.int32)
    dst2d = edge_index[1].astype(jnp.int32).reshape(_NCHUNK, _CHUNK)
    for l in range(L):
        e_l = _e_matmul(edge_attr, We[l], E)
        aggp = _sc_gin(h, e_l, src, dst2d)
        h = _gin_mlp(h, aggp, eps[l], W1[l], b1[l], W2[l], b2[l])

    return _global_pool(h, batch)


# speedup vs baseline: 12.4324x; 1.0011x over previous
import dataclasses
import functools

import jax
import jax.numpy as jnp
from jax import lax
from jax.experimental import pallas as pl
from jax.experimental.pallas import tpu as pltpu
from jax.experimental.pallas import tpu_sc as plsc

N = 10000
E = 320000
S = 40000
ES = 320000
M = 4
G = 8
D = 128
DE = 16
H = 128
L = 3


_ROW_BLK = 2048


def _mm_relu_body(x_ref, w_ref, b_ref, o_ref):
    o_ref[...] = jax.nn.relu(
        jnp.dot(x_ref[...], w_ref[...], preferred_element_type=jnp.float32)
        + b_ref[...])


def _mm_relu(x, w, b):
    n = x.shape[0]
    grid = (n + _ROW_BLK - 1) // _ROW_BLK
    kin = x.shape[1]
    return pl.pallas_call(
        _mm_relu_body,
        grid=(grid,),
        in_specs=[
            pl.BlockSpec((_ROW_BLK, kin), lambda i: (i, 0)),
            pl.BlockSpec((kin, H), lambda i: (0, 0)),
            pl.BlockSpec((1, H), lambda i: (0, 0)),
        ],
        out_specs=pl.BlockSpec((_ROW_BLK, H), lambda i: (i, 0)),
        out_shape=jax.ShapeDtypeStruct((n, H), jnp.float32),
    )(x, w, b.reshape(1, H))


def _gin_mlp_body(h_ref, agg_ref, eps_ref, w1_ref, b1_ref, w2_ref, b2_ref, o_ref):
    z = (1.0 + eps_ref[0, 0]) * h_ref[...] + agg_ref[0] + agg_ref[1]
    t = jax.nn.relu(
        jnp.dot(z, w1_ref[...], preferred_element_type=jnp.float32) + b1_ref[...])
    o_ref[...] = jax.nn.relu(
        jnp.dot(t, w2_ref[...], preferred_element_type=jnp.float32) + b2_ref[...])


def _gin_mlp(h, aggp, eps_l, w1, b1, w2, b2):
    grid = (N + _ROW_BLK - 1) // _ROW_BLK
    return pl.pallas_call(
        _gin_mlp_body,
        grid=(grid,),
        in_specs=[
            pl.BlockSpec((_ROW_BLK, H), lambda i: (i, 0)),
            pl.BlockSpec((2, _ROW_BLK, H), lambda i: (0, i, 0)),
            pl.BlockSpec((1, 1), lambda i: (0, 0)),
            pl.BlockSpec((H, H), lambda i: (0, 0)),
            pl.BlockSpec((1, H), lambda i: (0, 0)),
            pl.BlockSpec((H, H), lambda i: (0, 0)),
            pl.BlockSpec((1, H), lambda i: (0, 0)),
        ],
        out_specs=pl.BlockSpec((_ROW_BLK, H), lambda i: (i, 0)),
        out_shape=jax.ShapeDtypeStruct((N, H), jnp.float32),
    )(h, aggp, eps_l.reshape(1, 1), w1, b1.reshape(1, H), w2, b2.reshape(1, H))


_EBLK = 2048


def _e_body(a_ref, w_ref, o_ref):
    o_ref[...] = jnp.dot(a_ref[...], w_ref[...],
                         preferred_element_type=jnp.float32)


def _e_matmul(edge_attr, we_l, n_out):
    grid = (n_out + _EBLK - 1) // _EBLK
    return pl.pallas_call(
        _e_body,
        grid=(grid,),
        in_specs=[
            pl.BlockSpec((_EBLK, DE), lambda i: (i, 0)),
            pl.BlockSpec((DE, H), lambda i: (0, 0)),
        ],
        out_specs=pl.BlockSpec((_EBLK, H), lambda i: (i, 0)),
        out_shape=jax.ShapeDtypeStruct((n_out, H), jnp.float32),
    )(edge_attr, we_l)


_NBLK = 512


def _stage1_mlp_body(ssum_ref, tg_ref, st_ref, en_ref, x_ref,
                     ws2_ref, bs2_ref, wpt_ref, wpb_ref, bp_ref, o_ref):
    psum = ssum_ref[0] + ssum_ref[1]
    cnt = jnp.maximum((en_ref[...] - st_ref[...]).astype(jnp.float32), 1.0)
    pre = psum / cnt + tg_ref[...]
    se = jax.nn.relu(
        jnp.dot(pre, ws2_ref[...], preferred_element_type=jnp.float32)
        + bs2_ref[...])
    ne = se.reshape(_NBLK, M, H).mean(axis=1)
    o_ref[...] = jax.nn.relu(
        jnp.dot(x_ref[...], wpt_ref[...], preferred_element_type=jnp.float32)
        + jnp.dot(ne, wpb_ref[...], preferred_element_type=jnp.float32)
        + bp_ref[...])


def _stage1_mlp(ssum, tg, starts, ends, x, ws2, bs2, wp, bp):
    grid = (N + _NBLK - 1) // _NBLK
    wpt = wp[:D]
    wpb = wp[D:]
    sb = M * _NBLK
    return pl.pallas_call(
        _stage1_mlp_body,
        grid=(grid,),
        in_specs=[
            pl.BlockSpec((2, sb, H), lambda i: (0, i, 0)),
            pl.BlockSpec((sb, H), lambda i: (i, 0)),
            pl.BlockSpec((sb, 1), lambda i: (i, 0)),
            pl.BlockSpec((sb, 1), lambda i: (i, 0)),
            pl.BlockSpec((_NBLK, D), lambda i: (i, 0)),
            pl.BlockSpec((H, H), lambda i: (0, 0)),
            pl.BlockSpec((1, H), lambda i: (0, 0)),
            pl.BlockSpec((D, H), lambda i: (0, 0)),
            pl.BlockSpec((H, H), lambda i: (0, 0)),
            pl.BlockSpec((1, H), lambda i: (0, 0)),
        ],
        out_specs=pl.BlockSpec((_NBLK, H), lambda i: (i, 0)),
        out_shape=jax.ShapeDtypeStruct((N, H), jnp.float32),
    )(ssum, tg, starts, ends, x, ws2, bs2.reshape(1, H), wpt, wpb,
      bp.reshape(1, H))


_POOL_BLK = 1024


def _pool_body(h_ref, batch_ref, out_ref, acc_ref, cnt_ref):
    i = pl.program_id(0)

    @pl.when(i == 0)
    def _init():
        acc_ref[...] = jnp.zeros_like(acc_ref)
        cnt_ref[...] = jnp.zeros_like(cnt_ref)

    rows = h_ref[...]
    b = batch_ref[...]
    gids = lax.broadcasted_iota(jnp.int32, (G, _POOL_BLK), 0)
    oh = jnp.where(gids == b, 1.0, 0.0)
    acc_ref[...] += jnp.dot(oh, rows, preferred_element_type=jnp.float32)
    cnt_ref[...] += jnp.sum(oh, axis=1, keepdims=True)

    @pl.when(i == pl.num_programs(0) - 1)
    def _fin():
        out_ref[...] = acc_ref[...] / jnp.maximum(cnt_ref[...], 1.0)


def _global_pool(h, batch):
    n_pad = ((N + _POOL_BLK - 1) // _POOL_BLK) * _POOL_BLK
    h_p = jnp.pad(h, ((0, n_pad - N), (0, 0)))
    b_p = jnp.pad(batch.astype(jnp.int32), (0, n_pad - N),
                  constant_values=G).reshape(1, n_pad)
    grid = n_pad // _POOL_BLK
    return pl.pallas_call(
        _pool_body,
        grid=(grid,),
        in_specs=[
            pl.BlockSpec((_POOL_BLK, H), lambda i: (i, 0)),
            pl.BlockSpec((1, _POOL_BLK), lambda i: (0, i)),
        ],
        out_specs=pl.BlockSpec((G, H), lambda i: (0, 0)),
        out_shape=jax.ShapeDtypeStruct((G, H), jnp.float32),
        scratch_shapes=[
            pltpu.VMEM((G, H), jnp.float32),
            pltpu.VMEM((G, 1), jnp.float32),
        ],
    )(h_p, b_p)


_NC, _NS = 2, 16
_CHUNK = 128
_NCHUNK = E // _CHUNK
_CPT = (_NCHUNK + _NC * _NS - 1) // (_NC * _NS)
_NRB = (N + _CHUNK - 1) // _CHUNK
_RBPT = (_NRB + _NS - 1) // _NS
_NRB_FULL = N // _CHUNK


def _sc_gin_body(h_hbm, e_hbm, src_hbm, dst_hbm, out_hbm,
                 src_v, dst_v, hbuf, ebuf, agg):
    c = lax.axis_index("c")
    s = lax.axis_index("s")
    w = c * _NS + s

    @pl.loop(0, _CHUNK)
    def _z(r):
        for u in range(8):
            hbuf[r, pl.ds(u * 16, 16)] = jnp.zeros((16,), jnp.float32)

    tail = N - _NRB_FULL * _CHUNK

    @pl.loop(0, _RBPT)
    def _zb(k):
        z = s * _RBPT + k

        @pl.when(z < _NRB_FULL)
        def _full():
            pltpu.sync_copy(hbuf, agg.at[pl.ds(z * _CHUNK, _CHUNK)])

        @pl.when(z == _NRB_FULL)
        def _tail():
            pltpu.sync_copy(hbuf.at[pl.ds(0, tail)],
                            agg.at[pl.ds(_NRB_FULL * _CHUNK, tail)])

    plsc.subcore_barrier()

    @pl.loop(0, _CPT)
    def _chunk(ci):
        g = w * _CPT + ci

        @pl.when(g < _NCHUNK)
        def _():
            base = g * _CHUNK
            pltpu.sync_copy(src_hbm.at[pl.ds(base, _CHUNK)], src_v)
            pltpu.sync_copy(dst_hbm.at[g], dst_v.at[0])
            pltpu.sync_copy(h_hbm.at[src_v], hbuf)
            pltpu.sync_copy(e_hbm.at[pl.ds(base, _CHUNK)], ebuf)

            @pl.loop(0, _CHUNK, step=2)
            def _row(r):
                for rr in range(2):
                    for u in range(8):
                        sl = pl.ds(u * 16, 16)
                        ebuf[r + rr, sl] = jnp.maximum(
                            ebuf[r + rr, sl] + hbuf[r + rr, sl], 0.0)

            pltpu.sync_copy(ebuf, agg.at[dst_v.at[0]], add=True)

    plsc.subcore_barrier()

    @pl.loop(0, _RBPT)
    def _wb(k):
        z = s * _RBPT + k

        @pl.when(z < _NRB_FULL)
        def _full():
            pltpu.sync_copy(agg.at[pl.ds(z * _CHUNK, _CHUNK)],
                            out_hbm.at[c].at[pl.ds(z * _CHUNK, _CHUNK)])

        @pl.when(z == _NRB_FULL)
        def _tail():
            pltpu.sync_copy(agg.at[pl.ds(_NRB_FULL * _CHUNK, tail)],
                            out_hbm.at[c].at[pl.ds(_NRB_FULL * _CHUNK, tail)])


@functools.partial(
    pl.kernel,
    out_type=jax.ShapeDtypeStruct((_NC, N, H), jnp.float32),
    mesh=plsc.VectorSubcoreMesh(core_axis_name="c", subcore_axis_name="s"),
    scratch_types=[
        pltpu.VMEM((_CHUNK,), jnp.int32),
        pltpu.VMEM((1, _CHUNK), jnp.int32),
        pltpu.VMEM((_CHUNK, H), jnp.float32),
        pltpu.VMEM((_CHUNK, H), jnp.float32),
        pltpu.VMEM_SHARED((N, H), jnp.float32),
    ],
)
def _sc_gin(h_hbm, e_hbm, src_hbm, dst_hbm, out_hbm,
            src_v, dst_v, hbuf, ebuf, agg):
    _sc_gin_body(h_hbm, e_hbm, src_hbm, dst_hbm, out_hbm,
                 src_v, dst_v, hbuf, ebuf, agg)


_SC_CP = pltpu.CompilerParams()
if "needs_layout_passes" in pltpu.CompilerParams.__dataclass_fields__:
    _SC_CP = dataclasses.replace(_SC_CP, needs_layout_passes=False)

_PTR_PAD = 40016
_SBLK = 8000
_NPH = S // _SBLK
_S1_NFULL = _SBLK // _CHUNK
_S1_TAIL = _SBLK - _S1_NFULL * _CHUNK
_S1_NRB = _S1_NFULL + 1
_S1_RBPT = (_S1_NRB + _NS - 1) // _NS
_ECPT = _CPT
_TGC = (S + 64) // _CHUNK
_TGPT = (_TGC + _NC * _NS - 1) // (_NC * _NS)
_SEGLEN = _ECPT * _CHUNK


def _count_lt(ptrv, x):
    iota = lax.iota(jnp.int32, 16)
    base = jnp.int32(0)
    for stride in (2501, 157, 10, 1):
        idx = jnp.minimum(base + (iota + 1) * stride - 1, _PTR_PAD - 1)
        vals = plsc.load_gather(ptrv, [idx])
        cnt = jnp.sum((vals < x).astype(jnp.int32))
        base = base + cnt * stride
    return base


def _sc_stage1_body(h_hbm, esg_hbm, eis_hbm, ptr_hbm, ns_hbm,
                    ssum_hbm, tg_hbm,
                    ptrv, seg1d, adj, iv, abuf, agg):
    c = lax.axis_index("c")
    s = lax.axis_index("s")
    w = c * _NS + s
    iota = lax.iota(jnp.int32, 16)

    pltpu.sync_copy(ptr_hbm, ptrv)

    @pl.loop(0, _TGPT)
    def _tg(k):
        g = w * _TGPT + k

        @pl.when(g < _TGC)
        def _():
            pltpu.sync_copy(ns_hbm.at[g], iv)
            pltpu.sync_copy(h_hbm.at[iv], abuf)
            pltpu.sync_copy(abuf, tg_hbm.at[pl.ds(g * _CHUNK, _CHUNK)])

    b0 = w * _SEGLEN
    b1 = jnp.minimum(b0 + _SEGLEN, ES)
    sa = _count_lt(ptrv, b0)
    sb = _count_lt(ptrv, b1)
    seed = jnp.clip(sa - 1, 0, S - 1)

    @pl.loop(0, _ECPT)
    def _zs(ci):
        for u in range(8):
            seg1d[pl.ds(ci * _CHUNK + u * 16, 16)] = jnp.zeros((16,), jnp.int32)

    ngrp = (sb - sa + 15) // 16

    def _scatter_starts(gi, _):
        t16 = sa + gi * 16 + iota
        tc = jnp.minimum(t16, _PTR_PAD - 1)
        vals = plsc.load_gather(ptrv, [tc])
        nxt = plsc.load_gather(ptrv, [jnp.minimum(tc + 1, _PTR_PAD - 1)])
        pos = vals - b0
        keep = ((t16 < sb) & (vals != nxt)
                & (pos >= 0) & (pos < _SEGLEN))
        plsc.store_scatter(seg1d, [jnp.where(keep, pos, 0)],
                           jnp.minimum(t16, S - 1), mask=keep)
        return 0

    lax.fori_loop(0, ngrp, _scatter_starts, 0)

    def _sweep(gi, carry):
        v = seg1d[pl.ds(gi * 16, 16)]
        v = jnp.maximum(plsc.cummax(v), carry)
        seg1d[pl.ds(gi * 16, 16)] = v
        return jnp.full((16,), jnp.max(v), jnp.int32)

    lax.fori_loop(0, _SEGLEN // 16, _sweep,
                  jnp.full((16,), seed, jnp.int32))

    @pl.loop(0, _CHUNK)
    def _zb0(r):
        for u in range(8):
            abuf.at[r][pl.ds(u * 16, 16)] = jnp.zeros((16,), jnp.float32)

    tail = _S1_TAIL
    nfull = _S1_NFULL

    for ph in range(_NPH):
        blo = ph * _SBLK

        @pl.loop(0, _S1_RBPT)
        def _zb(k):
            z = s * _S1_RBPT + k

            @pl.when(z < nfull)
            def _full():
                pltpu.sync_copy(abuf, agg.at[pl.ds(z * _CHUNK, _CHUNK)])

            @pl.when(z == nfull)
            def _tl():
                pltpu.sync_copy(abuf.at[pl.ds(0, tail)],
                                agg.at[pl.ds(nfull * _CHUNK, tail)])

        plsc.subcore_barrier()

        @pl.loop(0, _ECPT)
        def _chunk(ci):
            g = w * _ECPT + ci

            @pl.when(g < _NCHUNK)
            def _():
                lo16 = seg1d[pl.ds(ci * _CHUNK, 16)]
                hi16 = seg1d[pl.ds(ci * _CHUNK + _CHUNK - 16, 16)]
                cmin = jnp.min(lo16)
                cmax = jnp.max(hi16)

                @pl.when((cmax >= blo) & (cmin < blo + _SBLK))
                def _active():
                    adj_row = adj.at[0]
                    for u in range(8):
                        v = seg1d[pl.ds(ci * _CHUNK + u * 16, 16)] - blo
                        inb = (v >= 0) & (v < _SBLK)
                        adj_row[pl.ds(u * 16, 16)] = jnp.where(inb, v, _SBLK)
                    base = g * _CHUNK
                    pltpu.sync_copy(esg_hbm.at[pl.ds(base, _CHUNK)], iv)
                    pltpu.sync_copy(h_hbm.at[iv], abuf)
                    pltpu.sync_copy(abuf, agg.at[adj.at[0]], add=True)
                    pltpu.sync_copy(eis_hbm.at[pl.ds(base, _CHUNK)], iv)
                    pltpu.sync_copy(h_hbm.at[iv], abuf)
                    pltpu.sync_copy(abuf, agg.at[adj.at[0]], add=True)

        plsc.subcore_barrier()

        @pl.loop(0, _S1_RBPT)
        def _wb(k):
            z = s * _S1_RBPT + k

            @pl.when(z < nfull)
            def _full():
                pltpu.sync_copy(
                    agg.at[pl.ds(z * _CHUNK, _CHUNK)],
                    ssum_hbm.at[c].at[pl.ds(blo + z * _CHUNK, _CHUNK)])

            @pl.when(z == nfull)
            def _tl():
                pltpu.sync_copy(
                    agg.at[pl.ds(nfull * _CHUNK, tail)],
                    ssum_hbm.at[c].at[pl.ds(blo + nfull * _CHUNK, tail)])

        if ph + 1 < _NPH:
            @pl.loop(0, _CHUNK)
            def _rz(r):
                for u in range(8):
                    abuf.at[r][pl.ds(u * 16, 16)] = jnp.zeros((16,), jnp.float32)


@functools.partial(
    pl.kernel,
    out_type=(jax.ShapeDtypeStruct((_NC, S, H), jnp.float32),
              jax.ShapeDtypeStruct((_TGC * _CHUNK, H), jnp.float32)),
    mesh=plsc.VectorSubcoreMesh(core_axis_name="c", subcore_axis_name="s"),
    scratch_types=[
        pltpu.VMEM((_PTR_PAD,), jnp.int32),
        pltpu.VMEM((_SEGLEN,), jnp.int32),
        pltpu.VMEM((1, _CHUNK), jnp.int32),
        pltpu.VMEM((_CHUNK,), jnp.int32),
        pltpu.VMEM((_CHUNK, H), jnp.float32),
        pltpu.VMEM_SHARED((_SBLK + 1, H), jnp.float32),
    ],
    compiler_params=_SC_CP,
)
def _sc_stage1(h_hbm, esg_hbm, eis_hbm, ptr_hbm, ns_hbm, ssum_hbm, tg_hbm,
               ptrv, seg1d, adj, iv, abuf, agg):
    _sc_stage1_body(h_hbm, esg_hbm, eis_hbm, ptr_hbm, ns_hbm,
                    ssum_hbm, tg_hbm, ptrv, seg1d, adj, iv, abuf, agg)


def kernel(x, edge_attr, nodes_sampled, edge_index_sampled, edge_ptr,
           edge_src_global, edge_index, batch,
           W_sub1, b_sub1, W_sub2, b_sub2, W_proj, b_proj,
           We, W1, b1, W2, b2, eps):
    h_all = _mm_relu(x, W_sub1, b_sub1)
    ptr32 = edge_ptr.astype(jnp.int32)
    ptr_pad = jnp.pad(ptr32, (0, _PTR_PAD - (S + 1)),
                      constant_values=jnp.int32(2**31 - 1))
    ns_pad = jnp.pad(nodes_sampled.astype(jnp.int32),
                     (0, _TGC * _CHUNK - S)).reshape(_TGC, _CHUNK)
    esg = edge_src_global.astype(jnp.int32)
    eis = edge_index_sampled[1].astype(jnp.int32)
    ssum, tg = _sc_stage1(h_all, esg, eis, ptr_pad, ns_pad)
    starts = jnp.concatenate([jnp.zeros((1,), jnp.int32),
                              ptr32[1:S]]).reshape(S, 1)
    ends = jnp.concatenate([ptr32[1:S],
                            jnp.full((1,), ES, jnp.int32)]).reshape(S, 1)
    h = _stage1_mlp(ssum, tg, starts, ends, x, W_sub2, b_sub2, W_proj, b_proj)

    src = edge_index[0].astype(jnp.int32)
    dst2d = edge_index[1].astype(jnp.int32).reshape(_NCHUNK, _CHUNK)
    for l in range(L):
        e_l = _e_matmul(edge_attr, We[l], E)
        aggp = _sc_gin(h, e_l, src, dst2d)
        h = _gin_mlp(h, aggp, eps[l], W1[l], b1[l], W2[l], b2[l])

    return _global_pool(h, batch)


# speedup vs baseline: 13.8422x; 1.1134x over previous
import dataclasses
import functools

import jax
import jax.numpy as jnp
from jax import lax
from jax.experimental import pallas as pl
from jax.experimental.pallas import tpu as pltpu
from jax.experimental.pallas import tpu_sc as plsc

N = 10000
E = 320000
S = 40000
ES = 320000
M = 4
G = 8
D = 128
DE = 16
H = 128
L = 3


_ROW_BLK = 2048


def _mm_relu_body(x_ref, w_ref, b_ref, o_ref):
    o_ref[...] = jax.nn.relu(
        jnp.dot(x_ref[...], w_ref[...], preferred_element_type=jnp.float32)
        + b_ref[...])


def _mm_relu(x, w, b):
    n = x.shape[0]
    grid = (n + _ROW_BLK - 1) // _ROW_BLK
    kin = x.shape[1]
    return pl.pallas_call(
        _mm_relu_body,
        grid=(grid,),
        in_specs=[
            pl.BlockSpec((_ROW_BLK, kin), lambda i: (i, 0)),
            pl.BlockSpec((kin, H), lambda i: (0, 0)),
            pl.BlockSpec((1, H), lambda i: (0, 0)),
        ],
        out_specs=pl.BlockSpec((_ROW_BLK, H), lambda i: (i, 0)),
        out_shape=jax.ShapeDtypeStruct((n, H), jnp.float32),
    )(x, w, b.reshape(1, H))


def _gin_mlp_body(h_ref, agg_ref, eps_ref, w1_ref, b1_ref, w2_ref, b2_ref, o_ref):
    z = (1.0 + eps_ref[0, 0]) * h_ref[...] + agg_ref[0] + agg_ref[1]
    t = jax.nn.relu(
        jnp.dot(z, w1_ref[...], preferred_element_type=jnp.float32) + b1_ref[...])
    o_ref[...] = jax.nn.relu(
        jnp.dot(t, w2_ref[...], preferred_element_type=jnp.float32) + b2_ref[...])


def _gin_mlp(h, aggp, eps_l, w1, b1, w2, b2):
    grid = (N + _ROW_BLK - 1) // _ROW_BLK
    return pl.pallas_call(
        _gin_mlp_body,
        grid=(grid,),
        in_specs=[
            pl.BlockSpec((_ROW_BLK, H), lambda i: (i, 0)),
            pl.BlockSpec((2, _ROW_BLK, H), lambda i: (0, i, 0)),
            pl.BlockSpec((1, 1), lambda i: (0, 0)),
            pl.BlockSpec((H, H), lambda i: (0, 0)),
            pl.BlockSpec((1, H), lambda i: (0, 0)),
            pl.BlockSpec((H, H), lambda i: (0, 0)),
            pl.BlockSpec((1, H), lambda i: (0, 0)),
        ],
        out_specs=pl.BlockSpec((_ROW_BLK, H), lambda i: (i, 0)),
        out_shape=jax.ShapeDtypeStruct((N, H), jnp.float32),
    )(h, aggp, eps_l.reshape(1, 1), w1, b1.reshape(1, H), w2, b2.reshape(1, H))


_EBLK = 2048


def _e_body(a_ref, w_ref, o_ref):
    o_ref[...] = jnp.dot(a_ref[...], w_ref[...],
                         preferred_element_type=jnp.float32)


def _e_matmul(edge_attr, we_l, n_out):
    grid = (n_out + _EBLK - 1) // _EBLK
    return pl.pallas_call(
        _e_body,
        grid=(grid,),
        in_specs=[
            pl.BlockSpec((_EBLK, DE), lambda i: (i, 0)),
            pl.BlockSpec((DE, H), lambda i: (0, 0)),
        ],
        out_specs=pl.BlockSpec((_EBLK, H), lambda i: (i, 0)),
        out_shape=jax.ShapeDtypeStruct((n_out, H), jnp.float32),
    )(edge_attr, we_l)


_NBLK = 512


def _stage1_mlp_body(ssum_ref, tg_ref, st_ref, en_ref, x_ref,
                     ws2_ref, bs2_ref, wpt_ref, wpb_ref, bp_ref, o_ref):
    psum = ssum_ref[0] + ssum_ref[1]
    cnt = jnp.maximum((en_ref[...] - st_ref[...]).astype(jnp.float32), 1.0)
    pre = psum / cnt + tg_ref[...]
    se = jax.nn.relu(
        jnp.dot(pre, ws2_ref[...], preferred_element_type=jnp.float32)
        + bs2_ref[...])
    ne = se.reshape(_NBLK, M, H).mean(axis=1)
    o_ref[...] = jax.nn.relu(
        jnp.dot(x_ref[...], wpt_ref[...], preferred_element_type=jnp.float32)
        + jnp.dot(ne, wpb_ref[...], preferred_element_type=jnp.float32)
        + bp_ref[...])


def _stage1_mlp(ssum, tg, starts, ends, x, ws2, bs2, wp, bp):
    grid = (N + _NBLK - 1) // _NBLK
    wpt = wp[:D]
    wpb = wp[D:]
    sb = M * _NBLK
    return pl.pallas_call(
        _stage1_mlp_body,
        grid=(grid,),
        in_specs=[
            pl.BlockSpec((2, sb, H), lambda i: (0, i, 0)),
            pl.BlockSpec((sb, H), lambda i: (i, 0)),
            pl.BlockSpec((sb, 1), lambda i: (i, 0)),
            pl.BlockSpec((sb, 1), lambda i: (i, 0)),
            pl.BlockSpec((_NBLK, D), lambda i: (i, 0)),
            pl.BlockSpec((H, H), lambda i: (0, 0)),
            pl.BlockSpec((1, H), lambda i: (0, 0)),
            pl.BlockSpec((D, H), lambda i: (0, 0)),
            pl.BlockSpec((H, H), lambda i: (0, 0)),
            pl.BlockSpec((1, H), lambda i: (0, 0)),
        ],
        out_specs=pl.BlockSpec((_NBLK, H), lambda i: (i, 0)),
        out_shape=jax.ShapeDtypeStruct((N, H), jnp.float32),
    )(ssum, tg, starts, ends, x, ws2, bs2.reshape(1, H), wpt, wpb,
      bp.reshape(1, H))


_POOL_BLK = 1024


def _pool_body(h_ref, batch_ref, out_ref, acc_ref, cnt_ref):
    i = pl.program_id(0)

    @pl.when(i == 0)
    def _init():
        acc_ref[...] = jnp.zeros_like(acc_ref)
        cnt_ref[...] = jnp.zeros_like(cnt_ref)

    rows = h_ref[...]
    b = batch_ref[...]
    gids = lax.broadcasted_iota(jnp.int32, (G, _POOL_BLK), 0)
    oh = jnp.where(gids == b, 1.0, 0.0)
    acc_ref[...] += jnp.dot(oh, rows, preferred_element_type=jnp.float32)
    cnt_ref[...] += jnp.sum(oh, axis=1, keepdims=True)

    @pl.when(i == pl.num_programs(0) - 1)
    def _fin():
        out_ref[...] = acc_ref[...] / jnp.maximum(cnt_ref[...], 1.0)


def _global_pool(h, batch):
    n_pad = ((N + _POOL_BLK - 1) // _POOL_BLK) * _POOL_BLK
    h_p = jnp.pad(h, ((0, n_pad - N), (0, 0)))
    b_p = jnp.pad(batch.astype(jnp.int32), (0, n_pad - N),
                  constant_values=G).reshape(1, n_pad)
    grid = n_pad // _POOL_BLK
    return pl.pallas_call(
        _pool_body,
        grid=(grid,),
        in_specs=[
            pl.BlockSpec((_POOL_BLK, H), lambda i: (i, 0)),
            pl.BlockSpec((1, _POOL_BLK), lambda i: (0, i)),
        ],
        out_specs=pl.BlockSpec((G, H), lambda i: (0, 0)),
        out_shape=jax.ShapeDtypeStruct((G, H), jnp.float32),
        scratch_shapes=[
            pltpu.VMEM((G, H), jnp.float32),
            pltpu.VMEM((G, 1), jnp.float32),
        ],
    )(h_p, b_p)


_NC, _NS = 2, 16
_CHUNK = 128
_NCHUNK = E // _CHUNK
_CPT = (_NCHUNK + _NC * _NS - 1) // (_NC * _NS)
_NRB = (N + _CHUNK - 1) // _CHUNK
_RBPT = (_NRB + _NS - 1) // _NS
_NRB_FULL = N // _CHUNK


def _sc_gin_body(h_hbm, e_hbm, src_hbm, dst_hbm, out_hbm,
                 src_v, dst_v, hbuf, ebuf, agg, m1, m2):
    c = lax.axis_index("c")
    s = lax.axis_index("s")
    w = c * _NS + s

    @pl.loop(0, _CHUNK)
    def _z(r):
        for u in range(8):
            hbuf[r, pl.ds(u * 16, 16)] = jnp.zeros((16,), jnp.float32)

    tail = N - _NRB_FULL * _CHUNK

    @pl.loop(0, _RBPT)
    def _zb(k):
        z = s * _RBPT + k

        @pl.when(z < _NRB_FULL)
        def _full():
            pltpu.sync_copy(hbuf, agg.at[pl.ds(z * _CHUNK, _CHUNK)])

        @pl.when(z == _NRB_FULL)
        def _tail():
            pltpu.sync_copy(hbuf.at[pl.ds(0, tail)],
                            agg.at[pl.ds(_NRB_FULL * _CHUNK, tail)])

    plsc.subcore_barrier()

    @pl.loop(0, _CPT)
    def _chunk(ci):
        g = w * _CPT + ci

        @pl.when(g < _NCHUNK)
        def _():
            base = g * _CHUNK
            d1 = pltpu.async_copy(src_hbm.at[pl.ds(base, _CHUNK)], src_v, m1)
            d2 = pltpu.async_copy(dst_hbm.at[g], dst_v.at[0], m2)
            d1.wait()
            d3 = pltpu.async_copy(h_hbm.at[src_v], hbuf, m1)
            d4 = pltpu.async_copy(e_hbm.at[pl.ds(base, _CHUNK)], ebuf, m2)
            d2.wait()
            d3.wait()
            d4.wait()

            @pl.loop(0, _CHUNK, step=2)
            def _row(r):
                for rr in range(2):
                    for u in range(8):
                        sl = pl.ds(u * 16, 16)
                        ebuf[r + rr, sl] = jnp.maximum(
                            ebuf[r + rr, sl] + hbuf[r + rr, sl], 0.0)

            pltpu.sync_copy(ebuf, agg.at[dst_v.at[0]], add=True)

    plsc.subcore_barrier()

    @pl.loop(0, _RBPT)
    def _wb(k):
        z = s * _RBPT + k

        @pl.when(z < _NRB_FULL)
        def _full():
            pltpu.sync_copy(agg.at[pl.ds(z * _CHUNK, _CHUNK)],
                            out_hbm.at[c].at[pl.ds(z * _CHUNK, _CHUNK)])

        @pl.when(z == _NRB_FULL)
        def _tail():
            pltpu.sync_copy(agg.at[pl.ds(_NRB_FULL * _CHUNK, tail)],
                            out_hbm.at[c].at[pl.ds(_NRB_FULL * _CHUNK, tail)])


@functools.partial(
    pl.kernel,
    out_type=jax.ShapeDtypeStruct((_NC, N, H), jnp.float32),
    mesh=plsc.VectorSubcoreMesh(core_axis_name="c", subcore_axis_name="s"),
    scratch_types=[
        pltpu.VMEM((_CHUNK,), jnp.int32),
        pltpu.VMEM((1, _CHUNK), jnp.int32),
        pltpu.VMEM((_CHUNK, H), jnp.float32),
        pltpu.VMEM((_CHUNK, H), jnp.float32),
        pltpu.VMEM_SHARED((N, H), jnp.float32),
        pltpu.SemaphoreType.DMA,
        pltpu.SemaphoreType.DMA,
    ],
)
def _sc_gin(h_hbm, e_hbm, src_hbm, dst_hbm, out_hbm,
            src_v, dst_v, hbuf, ebuf, agg, m1, m2):
    _sc_gin_body(h_hbm, e_hbm, src_hbm, dst_hbm, out_hbm,
                 src_v, dst_v, hbuf, ebuf, agg, m1, m2)


_SC_CP = pltpu.CompilerParams()
if "needs_layout_passes" in pltpu.CompilerParams.__dataclass_fields__:
    _SC_CP = dataclasses.replace(_SC_CP, needs_layout_passes=False)

_PTR_PAD = 40016
_SBLK = 8000
_NPH = S // _SBLK
_S1_NFULL = _SBLK // _CHUNK
_S1_TAIL = _SBLK - _S1_NFULL * _CHUNK
_S1_NRB = _S1_NFULL + 1
_S1_RBPT = (_S1_NRB + _NS - 1) // _NS
_ECPT = _CPT
_TGC = (S + 64) // _CHUNK
_TGPT = (_TGC + _NC * _NS - 1) // (_NC * _NS)
_SEGLEN = _ECPT * _CHUNK


def _count_lt(ptrv, x):
    iota = lax.iota(jnp.int32, 16)
    base = jnp.int32(0)
    for stride in (2501, 157, 10, 1):
        idx = jnp.minimum(base + (iota + 1) * stride - 1, _PTR_PAD - 1)
        vals = plsc.load_gather(ptrv, [idx])
        cnt = jnp.sum((vals < x).astype(jnp.int32))
        base = base + cnt * stride
    return base


def _sc_stage1_body(h_hbm, esg_hbm, eis_hbm, ptr_hbm, ns_hbm,
                    ssum_hbm, tg_hbm,
                    ptrv, seg1d, adj, iv, abuf, agg):
    c = lax.axis_index("c")
    s = lax.axis_index("s")
    w = c * _NS + s
    iota = lax.iota(jnp.int32, 16)

    pltpu.sync_copy(ptr_hbm, ptrv)

    @pl.loop(0, _TGPT)
    def _tg(k):
        g = w * _TGPT + k

        @pl.when(g < _TGC)
        def _():
            pltpu.sync_copy(ns_hbm.at[g], iv)
            pltpu.sync_copy(h_hbm.at[iv], abuf)
            pltpu.sync_copy(abuf, tg_hbm.at[pl.ds(g * _CHUNK, _CHUNK)])

    b0 = w * _SEGLEN
    b1 = jnp.minimum(b0 + _SEGLEN, ES)
    sa = _count_lt(ptrv, b0)
    sb = _count_lt(ptrv, b1)
    seed = jnp.clip(sa - 1, 0, S - 1)

    @pl.loop(0, _ECPT)
    def _zs(ci):
        for u in range(8):
            seg1d[pl.ds(ci * _CHUNK + u * 16, 16)] = jnp.zeros((16,), jnp.int32)

    ngrp = (sb - sa + 15) // 16

    def _scatter_starts(gi, _):
        t16 = sa + gi * 16 + iota
        tc = jnp.minimum(t16, _PTR_PAD - 1)
        vals = plsc.load_gather(ptrv, [tc])
        nxt = plsc.load_gather(ptrv, [jnp.minimum(tc + 1, _PTR_PAD - 1)])
        pos = vals - b0
        keep = ((t16 < sb) & (vals != nxt)
                & (pos >= 0) & (pos < _SEGLEN))
        plsc.store_scatter(seg1d, [jnp.where(keep, pos, 0)],
                           jnp.minimum(t16, S - 1), mask=keep)
        return 0

    lax.fori_loop(0, ngrp, _scatter_starts, 0)

    def _sweep(gi, carry):
        v = seg1d[pl.ds(gi * 16, 16)]
        v = jnp.maximum(plsc.cummax(v), carry)
        seg1d[pl.ds(gi * 16, 16)] = v
        return jnp.full((16,), jnp.max(v), jnp.int32)

    lax.fori_loop(0, _SEGLEN // 16, _sweep,
                  jnp.full((16,), seed, jnp.int32))

    @pl.loop(0, _CHUNK)
    def _zb0(r):
        for u in range(8):
            abuf.at[r][pl.ds(u * 16, 16)] = jnp.zeros((16,), jnp.float32)

    tail = _S1_TAIL
    nfull = _S1_NFULL

    for ph in range(_NPH):
        blo = ph * _SBLK

        @pl.loop(0, _S1_RBPT)
        def _zb(k):
            z = s * _S1_RBPT + k

            @pl.when(z < nfull)
            def _full():
                pltpu.sync_copy(abuf, agg.at[pl.ds(z * _CHUNK, _CHUNK)])

            @pl.when(z == nfull)
            def _tl():
                pltpu.sync_copy(abuf.at[pl.ds(0, tail)],
                                agg.at[pl.ds(nfull * _CHUNK, tail)])

        plsc.subcore_barrier()

        @pl.loop(0, _ECPT)
        def _chunk(ci):
            g = w * _ECPT + ci

            @pl.when(g < _NCHUNK)
            def _():
                lo16 = seg1d[pl.ds(ci * _CHUNK, 16)]
                hi16 = seg1d[pl.ds(ci * _CHUNK + _CHUNK - 16, 16)]
                cmin = jnp.min(lo16)
                cmax = jnp.max(hi16)

                @pl.when((cmax >= blo) & (cmin < blo + _SBLK))
                def _active():
                    adj_row = adj.at[0]
                    for u in range(8):
                        v = seg1d[pl.ds(ci * _CHUNK + u * 16, 16)] - blo
                        inb = (v >= 0) & (v < _SBLK)
                        adj_row[pl.ds(u * 16, 16)] = jnp.where(inb, v, _SBLK)
                    base = g * _CHUNK
                    pltpu.sync_copy(esg_hbm.at[pl.ds(base, _CHUNK)], iv)
                    pltpu.sync_copy(h_hbm.at[iv], abuf)
                    pltpu.sync_copy(abuf, agg.at[adj.at[0]], add=True)
                    pltpu.sync_copy(eis_hbm.at[pl.ds(base, _CHUNK)], iv)
                    pltpu.sync_copy(h_hbm.at[iv], abuf)
                    pltpu.sync_copy(abuf, agg.at[adj.at[0]], add=True)

        plsc.subcore_barrier()

        @pl.loop(0, _S1_RBPT)
        def _wb(k):
            z = s * _S1_RBPT + k

            @pl.when(z < nfull)
            def _full():
                pltpu.sync_copy(
                    agg.at[pl.ds(z * _CHUNK, _CHUNK)],
                    ssum_hbm.at[c].at[pl.ds(blo + z * _CHUNK, _CHUNK)])

            @pl.when(z == nfull)
            def _tl():
                pltpu.sync_copy(
                    agg.at[pl.ds(nfull * _CHUNK, tail)],
                    ssum_hbm.at[c].at[pl.ds(blo + nfull * _CHUNK, tail)])

        if ph + 1 < _NPH:
            @pl.loop(0, _CHUNK)
            def _rz(r):
                for u in range(8):
                    abuf.at[r][pl.ds(u * 16, 16)] = jnp.zeros((16,), jnp.float32)


@functools.partial(
    pl.kernel,
    out_type=(jax.ShapeDtypeStruct((_NC, S, H), jnp.float32),
              jax.ShapeDtypeStruct((_TGC * _CHUNK, H), jnp.float32)),
    mesh=plsc.VectorSubcoreMesh(core_axis_name="c", subcore_axis_name="s"),
    scratch_types=[
        pltpu.VMEM((_PTR_PAD,), jnp.int32),
        pltpu.VMEM((_SEGLEN,), jnp.int32),
        pltpu.VMEM((1, _CHUNK), jnp.int32),
        pltpu.VMEM((_CHUNK,), jnp.int32),
        pltpu.VMEM((_CHUNK, H), jnp.float32),
        pltpu.VMEM_SHARED((_SBLK + 1, H), jnp.float32),
    ],
    compiler_params=_SC_CP,
)
def _sc_stage1(h_hbm, esg_hbm, eis_hbm, ptr_hbm, ns_hbm, ssum_hbm, tg_hbm,
               ptrv, seg1d, adj, iv, abuf, agg):
    _sc_stage1_body(h_hbm, esg_hbm, eis_hbm, ptr_hbm, ns_hbm,
                    ssum_hbm, tg_hbm, ptrv, seg1d, adj, iv, abuf, agg)


def kernel(x, edge_attr, nodes_sampled, edge_index_sampled, edge_ptr,
           edge_src_global, edge_index, batch,
           W_sub1, b_sub1, W_sub2, b_sub2, W_proj, b_proj,
           We, W1, b1, W2, b2, eps):
    h_all = _mm_relu(x, W_sub1, b_sub1)
    ptr32 = edge_ptr.astype(jnp.int32)
    ptr_pad = jnp.pad(ptr32, (0, _PTR_PAD - (S + 1)),
                      constant_values=jnp.int32(2**31 - 1))
    ns_pad = jnp.pad(nodes_sampled.astype(jnp.int32),
                     (0, _TGC * _CHUNK - S)).reshape(_TGC, _CHUNK)
    esg = edge_src_global.astype(jnp.int32)
    eis = edge_index_sampled[1].astype(jnp.int32)
    ssum, tg = _sc_stage1(h_all, esg, eis, ptr_pad, ns_pad)
    starts = jnp.concatenate([jnp.zeros((1,), jnp.int32),
                              ptr32[1:S]]).reshape(S, 1)
    ends = jnp.concatenate([ptr32[1:S],
                            jnp.full((1,), ES, jnp.int32)]).reshape(S, 1)
    h = _stage1_mlp(ssum, tg, starts, ends, x, W_sub2, b_sub2, W_proj, b_proj)

    src = edge_index[0].astype(jnp.int32)
    dst2d = edge_index[1].astype(jnp.int32).reshape(_NCHUNK, _CHUNK)
    for l in range(L):
        e_l = _e_matmul(edge_attr, We[l], E)
        aggp = _sc_gin(h, e_l, src, dst2d)
        h = _gin_mlp(h, aggp, eps[l], W1[l], b1[l], W2[l], b2[l])

    return _global_pool(h, batch)


# speedup vs baseline: 14.5196x; 1.0489x over previous
import dataclasses
import functools

import jax
import jax.numpy as jnp
from jax import lax
from jax.experimental import pallas as pl
from jax.experimental.pallas import tpu as pltpu
from jax.experimental.pallas import tpu_sc as plsc

N = 10000
E = 320000
S = 40000
ES = 320000
M = 4
G = 8
D = 128
DE = 16
H = 128
L = 3


_ROW_BLK = 2048


def _mm_relu_body(x_ref, w_ref, b_ref, o_ref):
    o_ref[...] = jax.nn.relu(
        jnp.dot(x_ref[...], w_ref[...], preferred_element_type=jnp.float32)
        + b_ref[...])


def _mm_relu(x, w, b):
    n = x.shape[0]
    grid = (n + _ROW_BLK - 1) // _ROW_BLK
    kin = x.shape[1]
    return pl.pallas_call(
        _mm_relu_body,
        grid=(grid,),
        in_specs=[
            pl.BlockSpec((_ROW_BLK, kin), lambda i: (i, 0)),
            pl.BlockSpec((kin, H), lambda i: (0, 0)),
            pl.BlockSpec((1, H), lambda i: (0, 0)),
        ],
        out_specs=pl.BlockSpec((_ROW_BLK, H), lambda i: (i, 0)),
        out_shape=jax.ShapeDtypeStruct((n, H), jnp.float32),
    )(x, w, b.reshape(1, H))


def _gin_mlp_body(h_ref, agg_ref, eps_ref, w1_ref, b1_ref, w2_ref, b2_ref, o_ref):
    z = (1.0 + eps_ref[0, 0]) * h_ref[...] + agg_ref[0] + agg_ref[1]
    t = jax.nn.relu(
        jnp.dot(z, w1_ref[...], preferred_element_type=jnp.float32) + b1_ref[...])
    o_ref[...] = jax.nn.relu(
        jnp.dot(t, w2_ref[...], preferred_element_type=jnp.float32) + b2_ref[...])


def _gin_mlp(h, aggp, eps_l, w1, b1, w2, b2):
    grid = (N + _ROW_BLK - 1) // _ROW_BLK
    return pl.pallas_call(
        _gin_mlp_body,
        grid=(grid,),
        in_specs=[
            pl.BlockSpec((_ROW_BLK, H), lambda i: (i, 0)),
            pl.BlockSpec((2, _ROW_BLK, H), lambda i: (0, i, 0)),
            pl.BlockSpec((1, 1), lambda i: (0, 0)),
            pl.BlockSpec((H, H), lambda i: (0, 0)),
            pl.BlockSpec((1, H), lambda i: (0, 0)),
            pl.BlockSpec((H, H), lambda i: (0, 0)),
            pl.BlockSpec((1, H), lambda i: (0, 0)),
        ],
        out_specs=pl.BlockSpec((_ROW_BLK, H), lambda i: (i, 0)),
        out_shape=jax.ShapeDtypeStruct((N, H), jnp.float32),
    )(h, aggp, eps_l.reshape(1, 1), w1, b1.reshape(1, H), w2, b2.reshape(1, H))


_EBLK = 2048


def _e_body(a_ref, w_ref, o_ref):
    o_ref[...] = jnp.dot(a_ref[...], w_ref[...],
                         preferred_element_type=jnp.float32)


def _e_matmul(edge_attr, we_l, n_out):
    grid = (n_out + _EBLK - 1) // _EBLK
    return pl.pallas_call(
        _e_body,
        grid=(grid,),
        in_specs=[
            pl.BlockSpec((_EBLK, DE), lambda i: (i, 0)),
            pl.BlockSpec((DE, H), lambda i: (0, 0)),
        ],
        out_specs=pl.BlockSpec((_EBLK, H), lambda i: (i, 0)),
        out_shape=jax.ShapeDtypeStruct((n_out, H), jnp.float32),
    )(edge_attr, we_l)


_NBLK = 512


def _stage1_mlp_body(ssum_ref, tg_ref, st_ref, en_ref, x_ref,
                     ws2_ref, bs2_ref, wpt_ref, wpb_ref, bp_ref, o_ref):
    psum = ssum_ref[0] + ssum_ref[1]
    cnt = jnp.maximum((en_ref[...] - st_ref[...]).astype(jnp.float32), 1.0)
    pre = psum / cnt + tg_ref[...]
    se = jax.nn.relu(
        jnp.dot(pre, ws2_ref[...], preferred_element_type=jnp.float32)
        + bs2_ref[...])
    ne = se.reshape(_NBLK, M, H).mean(axis=1)
    o_ref[...] = jax.nn.relu(
        jnp.dot(x_ref[...], wpt_ref[...], preferred_element_type=jnp.float32)
        + jnp.dot(ne, wpb_ref[...], preferred_element_type=jnp.float32)
        + bp_ref[...])


def _stage1_mlp(ssum, tg, starts, ends, x, ws2, bs2, wp, bp):
    grid = (N + _NBLK - 1) // _NBLK
    wpt = wp[:D]
    wpb = wp[D:]
    sb = M * _NBLK
    return pl.pallas_call(
        _stage1_mlp_body,
        grid=(grid,),
        in_specs=[
            pl.BlockSpec((2, sb, H), lambda i: (0, i, 0)),
            pl.BlockSpec((sb, H), lambda i: (i, 0)),
            pl.BlockSpec((sb, 1), lambda i: (i, 0)),
            pl.BlockSpec((sb, 1), lambda i: (i, 0)),
            pl.BlockSpec((_NBLK, D), lambda i: (i, 0)),
            pl.BlockSpec((H, H), lambda i: (0, 0)),
            pl.BlockSpec((1, H), lambda i: (0, 0)),
            pl.BlockSpec((D, H), lambda i: (0, 0)),
            pl.BlockSpec((H, H), lambda i: (0, 0)),
            pl.BlockSpec((1, H), lambda i: (0, 0)),
        ],
        out_specs=pl.BlockSpec((_NBLK, H), lambda i: (i, 0)),
        out_shape=jax.ShapeDtypeStruct((N, H), jnp.float32),
    )(ssum, tg, starts, ends, x, ws2, bs2.reshape(1, H), wpt, wpb,
      bp.reshape(1, H))


_POOL_BLK = 1024


def _pool_body(h_ref, batch_ref, out_ref, acc_ref, cnt_ref):
    i = pl.program_id(0)

    @pl.when(i == 0)
    def _init():
        acc_ref[...] = jnp.zeros_like(acc_ref)
        cnt_ref[...] = jnp.zeros_like(cnt_ref)

    rows = h_ref[...]
    b = batch_ref[...]
    gids = lax.broadcasted_iota(jnp.int32, (G, _POOL_BLK), 0)
    oh = jnp.where(gids == b, 1.0, 0.0)
    acc_ref[...] += jnp.dot(oh, rows, preferred_element_type=jnp.float32)
    cnt_ref[...] += jnp.sum(oh, axis=1, keepdims=True)

    @pl.when(i == pl.num_programs(0) - 1)
    def _fin():
        out_ref[...] = acc_ref[...] / jnp.maximum(cnt_ref[...], 1.0)


def _global_pool(h, batch):
    n_pad = ((N + _POOL_BLK - 1) // _POOL_BLK) * _POOL_BLK
    h_p = jnp.pad(h, ((0, n_pad - N), (0, 0)))
    b_p = jnp.pad(batch.astype(jnp.int32), (0, n_pad - N),
                  constant_values=G).reshape(1, n_pad)
    grid = n_pad // _POOL_BLK
    return pl.pallas_call(
        _pool_body,
        grid=(grid,),
        in_specs=[
            pl.BlockSpec((_POOL_BLK, H), lambda i: (i, 0)),
            pl.BlockSpec((1, _POOL_BLK), lambda i: (0, i)),
        ],
        out_specs=pl.BlockSpec((G, H), lambda i: (0, 0)),
        out_shape=jax.ShapeDtypeStruct((G, H), jnp.float32),
        scratch_shapes=[
            pltpu.VMEM((G, H), jnp.float32),
            pltpu.VMEM((G, 1), jnp.float32),
        ],
    )(h_p, b_p)


_NC, _NS = 2, 16
_CHUNK = 128
_NCHUNK = E // _CHUNK
_CPT = (_NCHUNK + _NC * _NS - 1) // (_NC * _NS)
_NRB = (N + _CHUNK - 1) // _CHUNK
_RBPT = (_NRB + _NS - 1) // _NS
_NRB_FULL = N // _CHUNK


def _sc_gin_body(h_hbm, e_hbm, src_hbm, dst_hbm, out_hbm,
                 src_v, dst_v, hbuf, ebuf, agg, m1, m2):
    c = lax.axis_index("c")
    s = lax.axis_index("s")
    w = c * _NS + s

    @pl.loop(0, _CHUNK)
    def _z(r):
        for u in range(8):
            hbuf[r, pl.ds(u * 16, 16)] = jnp.zeros((16,), jnp.float32)

    tail = N - _NRB_FULL * _CHUNK

    @pl.loop(0, _RBPT)
    def _zb(k):
        z = s * _RBPT + k

        @pl.when(z < _NRB_FULL)
        def _full():
            pltpu.sync_copy(hbuf, agg.at[pl.ds(z * _CHUNK, _CHUNK)])

        @pl.when(z == _NRB_FULL)
        def _tail():
            pltpu.sync_copy(hbuf.at[pl.ds(0, tail)],
                            agg.at[pl.ds(_NRB_FULL * _CHUNK, tail)])

    plsc.subcore_barrier()

    @pl.loop(0, _CPT)
    def _chunk(ci):
        g = w * _CPT + ci

        @pl.when(g < _NCHUNK)
        def _():
            base = g * _CHUNK
            d1 = pltpu.async_copy(src_hbm.at[pl.ds(base, _CHUNK)], src_v, m1)
            d2 = pltpu.async_copy(dst_hbm.at[g], dst_v.at[0], m2)
            d1.wait()
            d3 = pltpu.async_copy(h_hbm.at[src_v], hbuf, m1)
            d4 = pltpu.async_copy(e_hbm.at[pl.ds(base, _CHUNK)], ebuf, m2)
            d2.wait()
            d3.wait()
            d4.wait()

            @pl.loop(0, _CHUNK, step=2)
            def _row(r):
                for rr in range(2):
                    for u in range(8):
                        sl = pl.ds(u * 16, 16)
                        ebuf[r + rr, sl] = jnp.maximum(
                            ebuf[r + rr, sl] + hbuf[r + rr, sl], 0.0)

            pltpu.sync_copy(ebuf, agg.at[dst_v.at[0]], add=True)

    plsc.subcore_barrier()

    @pl.loop(0, _RBPT)
    def _wb(k):
        z = s * _RBPT + k

        @pl.when(z < _NRB_FULL)
        def _full():
            pltpu.sync_copy(agg.at[pl.ds(z * _CHUNK, _CHUNK)],
                            out_hbm.at[c].at[pl.ds(z * _CHUNK, _CHUNK)])

        @pl.when(z == _NRB_FULL)
        def _tail():
            pltpu.sync_copy(agg.at[pl.ds(_NRB_FULL * _CHUNK, tail)],
                            out_hbm.at[c].at[pl.ds(_NRB_FULL * _CHUNK, tail)])


@functools.partial(
    pl.kernel,
    out_type=jax.ShapeDtypeStruct((_NC, N, H), jnp.float32),
    mesh=plsc.VectorSubcoreMesh(core_axis_name="c", subcore_axis_name="s"),
    scratch_types=[
        pltpu.VMEM((_CHUNK,), jnp.int32),
        pltpu.VMEM((1, _CHUNK), jnp.int32),
        pltpu.VMEM((_CHUNK, H), jnp.float32),
        pltpu.VMEM((_CHUNK, H), jnp.float32),
        pltpu.VMEM_SHARED((N, H), jnp.float32),
        pltpu.SemaphoreType.DMA,
        pltpu.SemaphoreType.DMA,
    ],
)
def _sc_gin(h_hbm, e_hbm, src_hbm, dst_hbm, out_hbm,
            src_v, dst_v, hbuf, ebuf, agg, m1, m2):
    _sc_gin_body(h_hbm, e_hbm, src_hbm, dst_hbm, out_hbm,
                 src_v, dst_v, hbuf, ebuf, agg, m1, m2)


_SC_CP = pltpu.CompilerParams()
if "needs_layout_passes" in pltpu.CompilerParams.__dataclass_fields__:
    _SC_CP = dataclasses.replace(_SC_CP, needs_layout_passes=False)

_PTR_PAD = 40016
_SBLK = 8000
_NPH = S // _SBLK
_S1_NFULL = _SBLK // _CHUNK
_S1_TAIL = _SBLK - _S1_NFULL * _CHUNK
_S1_NRB = _S1_NFULL + 1
_S1_RBPT = (_S1_NRB + _NS - 1) // _NS
_ECPT = _CPT
_TGC = (S + 64) // _CHUNK
_TGPT = (_TGC + _NC * _NS - 1) // (_NC * _NS)
_SEGLEN = _ECPT * _CHUNK


def _count_lt(ptrv, x):
    iota = lax.iota(jnp.int32, 16)
    base = jnp.int32(0)
    for stride in (2501, 157, 10, 1):
        idx = jnp.minimum(base + (iota + 1) * stride - 1, _PTR_PAD - 1)
        vals = plsc.load_gather(ptrv, [idx])
        cnt = jnp.sum((vals < x).astype(jnp.int32))
        base = base + cnt * stride
    return base


def _sc_stage1_body(h_hbm, esg_hbm, eis_hbm, ptr_hbm, ns_hbm,
                    ssum_hbm, tg_hbm,
                    ptrv, seg1d, adj, iv, iv2, abuf, agg, m1, m2):
    c = lax.axis_index("c")
    s = lax.axis_index("s")
    w = c * _NS + s
    iota = lax.iota(jnp.int32, 16)

    pltpu.sync_copy(ptr_hbm, ptrv)

    @pl.loop(0, _TGPT)
    def _tg(k):
        g = w * _TGPT + k

        @pl.when(g < _TGC)
        def _():
            pltpu.sync_copy(ns_hbm.at[g], iv)
            pltpu.sync_copy(h_hbm.at[iv], abuf)
            pltpu.sync_copy(abuf, tg_hbm.at[pl.ds(g * _CHUNK, _CHUNK)])

    b0 = w * _SEGLEN
    b1 = jnp.minimum(b0 + _SEGLEN, ES)
    sa = _count_lt(ptrv, b0)
    sb = _count_lt(ptrv, b1)
    seed = jnp.clip(sa - 1, 0, S - 1)

    @pl.loop(0, _ECPT)
    def _zs(ci):
        for u in range(8):
            seg1d[pl.ds(ci * _CHUNK + u * 16, 16)] = jnp.zeros((16,), jnp.int32)

    ngrp = (sb - sa + 15) // 16

    def _scatter_starts(gi, _):
        t16 = sa + gi * 16 + iota
        tc = jnp.minimum(t16, _PTR_PAD - 1)
        vals = plsc.load_gather(ptrv, [tc])
        nxt = plsc.load_gather(ptrv, [jnp.minimum(tc + 1, _PTR_PAD - 1)])
        pos = vals - b0
        keep = ((t16 < sb) & (vals != nxt)
                & (pos >= 0) & (pos < _SEGLEN))
        plsc.store_scatter(seg1d, [jnp.where(keep, pos, 0)],
                           jnp.minimum(t16, S - 1), mask=keep)
        return 0

    lax.fori_loop(0, ngrp, _scatter_starts, 0)

    def _sweep(gi, carry):
        v = seg1d[pl.ds(gi * 16, 16)]
        v = jnp.maximum(plsc.cummax(v), carry)
        seg1d[pl.ds(gi * 16, 16)] = v
        return jnp.full((16,), jnp.max(v), jnp.int32)

    lax.fori_loop(0, _SEGLEN // 16, _sweep,
                  jnp.full((16,), seed, jnp.int32))

    @pl.loop(0, _CHUNK)
    def _zb0(r):
        for u in range(8):
            abuf.at[r][pl.ds(u * 16, 16)] = jnp.zeros((16,), jnp.float32)

    tail = _S1_TAIL
    nfull = _S1_NFULL

    for ph in range(_NPH):
        blo = ph * _SBLK

        @pl.loop(0, _S1_RBPT)
        def _zb(k):
            z = s * _S1_RBPT + k

            @pl.when(z < nfull)
            def _full():
                pltpu.sync_copy(abuf, agg.at[pl.ds(z * _CHUNK, _CHUNK)])

            @pl.when(z == nfull)
            def _tl():
                pltpu.sync_copy(abuf.at[pl.ds(0, tail)],
                                agg.at[pl.ds(nfull * _CHUNK, tail)])

        plsc.subcore_barrier()

        @pl.loop(0, _ECPT)
        def _chunk(ci):
            g = w * _ECPT + ci

            @pl.when(g < _NCHUNK)
            def _():
                lo16 = seg1d[pl.ds(ci * _CHUNK, 16)]
                hi16 = seg1d[pl.ds(ci * _CHUNK + _CHUNK - 16, 16)]
                cmin = jnp.min(lo16)
                cmax = jnp.max(hi16)

                @pl.when((cmax >= blo) & (cmin < blo + _SBLK))
                def _active():
                    base = g * _CHUNK
                    d1 = pltpu.async_copy(esg_hbm.at[pl.ds(base, _CHUNK)],
                                          iv, m1)
                    d2 = pltpu.async_copy(eis_hbm.at[pl.ds(base, _CHUNK)],
                                          iv2, m2)
                    adj_row = adj.at[0]
                    for u in range(8):
                        v = seg1d[pl.ds(ci * _CHUNK + u * 16, 16)] - blo
                        inb = (v >= 0) & (v < _SBLK)
                        adj_row[pl.ds(u * 16, 16)] = jnp.where(inb, v, _SBLK)
                    d1.wait()
                    pltpu.sync_copy(h_hbm.at[iv], abuf)
                    pltpu.sync_copy(abuf, agg.at[adj.at[0]], add=True)
                    d2.wait()
                    pltpu.sync_copy(h_hbm.at[iv2], abuf)
                    pltpu.sync_copy(abuf, agg.at[adj.at[0]], add=True)

        plsc.subcore_barrier()

        @pl.loop(0, _S1_RBPT)
        def _wb(k):
            z = s * _S1_RBPT + k

            @pl.when(z < nfull)
            def _full():
                pltpu.sync_copy(
                    agg.at[pl.ds(z * _CHUNK, _CHUNK)],
                    ssum_hbm.at[c].at[pl.ds(blo + z * _CHUNK, _CHUNK)])

            @pl.when(z == nfull)
            def _tl():
                pltpu.sync_copy(
                    agg.at[pl.ds(nfull * _CHUNK, tail)],
                    ssum_hbm.at[c].at[pl.ds(blo + nfull * _CHUNK, tail)])

        if ph + 1 < _NPH:
            @pl.loop(0, _CHUNK)
            def _rz(r):
                for u in range(8):
                    abuf.at[r][pl.ds(u * 16, 16)] = jnp.zeros((16,), jnp.float32)


@functools.partial(
    pl.kernel,
    out_type=(jax.ShapeDtypeStruct((_NC, S, H), jnp.float32),
              jax.ShapeDtypeStruct((_TGC * _CHUNK, H), jnp.float32)),
    mesh=plsc.VectorSubcoreMesh(core_axis_name="c", subcore_axis_name="s"),
    scratch_types=[
        pltpu.VMEM((_PTR_PAD,), jnp.int32),
        pltpu.VMEM((_SEGLEN,), jnp.int32),
        pltpu.VMEM((1, _CHUNK), jnp.int32),
        pltpu.VMEM((_CHUNK,), jnp.int32),
        pltpu.VMEM((_CHUNK,), jnp.int32),
        pltpu.VMEM((_CHUNK, H), jnp.float32),
        pltpu.VMEM_SHARED((_SBLK + 1, H), jnp.float32),
        pltpu.SemaphoreType.DMA,
        pltpu.SemaphoreType.DMA,
    ],
    compiler_params=_SC_CP,
)
def _sc_stage1(h_hbm, esg_hbm, eis_hbm, ptr_hbm, ns_hbm, ssum_hbm, tg_hbm,
               ptrv, seg1d, adj, iv, iv2, abuf, agg, m1, m2):
    _sc_stage1_body(h_hbm, esg_hbm, eis_hbm, ptr_hbm, ns_hbm,
                    ssum_hbm, tg_hbm, ptrv, seg1d, adj, iv, iv2, abuf, agg,
                    m1, m2)


def kernel(x, edge_attr, nodes_sampled, edge_index_sampled, edge_ptr,
           edge_src_global, edge_index, batch,
           W_sub1, b_sub1, W_sub2, b_sub2, W_proj, b_proj,
           We, W1, b1, W2, b2, eps):
    h_all = _mm_relu(x, W_sub1, b_sub1)
    ptr32 = edge_ptr.astype(jnp.int32)
    ptr_pad = jnp.pad(ptr32, (0, _PTR_PAD - (S + 1)),
                      constant_values=jnp.int32(2**31 - 1))
    ns_pad = jnp.pad(nodes_sampled.astype(jnp.int32),
                     (0, _TGC * _CHUNK - S)).reshape(_TGC, _CHUNK)
    esg = edge_src_global.astype(jnp.int32)
    eis = edge_index_sampled[1].astype(jnp.int32)
    ssum, tg = _sc_stage1(h_all, esg, eis, ptr_pad, ns_pad)
    starts = jnp.concatenate([jnp.zeros((1,), jnp.int32),
                              ptr32[1:S]]).reshape(S, 1)
    ends = jnp.concatenate([ptr32[1:S],
                            jnp.full((1,), ES, jnp.int32)]).reshape(S, 1)
    h = _stage1_mlp(ssum, tg, starts, ends, x, W_sub2, b_sub2, W_proj, b_proj)

    src = edge_index[0].astype(jnp.int32)
    dst2d = edge_index[1].astype(jnp.int32).reshape(_NCHUNK, _CHUNK)
    for l in range(L):
        e_l = _e_matmul(edge_attr, We[l], E)
        aggp = _sc_gin(h, e_l, src, dst2d)
        h = _gin_mlp(h, aggp, eps[l], W1[l], b1[l], W2[l], b2[l])

    return _global_pool(h, batch)
